# baseline (speedup 1.0000x reference)
import functools

import jax
import jax.numpy as jnp
from jax.experimental import pallas as pl
from jax.experimental.pallas import tpu as pltpu

S, D = 2048, 1024
H, KV, HD = 16, 8, 64
E, TOPK, F = 8, 2, 1024
L = 2
EPS = 1e-6
THETA = 10000.0
HH = HD // 2

QW = H * HH
KW = KV * HH

BSQ = 512
NSB = S // BSQ

A = TOPK * S
BLK = 256
NB = A // BLK
G = NB + E - 1
A_PAD = G * BLK

SC_NC, SC_NS = 2, 16
SC_NW = SC_NC * SC_NS

_F32 = jnp.float32
_HI = jax.lax.Precision.HIGHEST


def _rms(x, w):
    return x * jax.lax.rsqrt(jnp.mean(x * x, axis=-1, keepdims=True) + EPS) * w


def _qkv_body(h_ref, ln1_ref, wq_ref, wk_ref, wv_ref, cq_ref, sq_ref,
              ck_ref, sk_ref, q_out, k_out, v_out):
    r = _rms(h_ref[...], ln1_ref[...])
    q = jnp.dot(r, wq_ref[...], preferred_element_type=_F32)
    k = jnp.dot(r, wk_ref[...], preferred_element_type=_F32)
    v = jnp.dot(r, wv_ref[...], preferred_element_type=_F32)
    qr = jnp.concatenate([-q[:, QW:], q[:, :QW]], axis=1)
    kr = jnp.concatenate([-k[:, KW:], k[:, :KW]], axis=1)
    q_out[...] = q * cq_ref[...] + qr * sq_ref[...]
    k_out[...] = k * ck_ref[...] + kr * sk_ref[...]
    v_out[...] = v


def _qkv_call(h, ln1, wq_p, wk_p, wv, cq, sq, ck, sk):
    return pl.pallas_call(
        _qkv_body,
        grid=(NSB,),
        in_specs=[
            pl.BlockSpec((BSQ, D), lambda i: (i, 0)),
            pl.BlockSpec((D,), lambda i: (0,)),
            pl.BlockSpec((D, 2 * QW), lambda i: (0, 0)),
            pl.BlockSpec((D, 2 * KW), lambda i: (0, 0)),
            pl.BlockSpec((D, KV * HD), lambda i: (0, 0)),
            pl.BlockSpec((BSQ, 2 * QW), lambda i: (i, 0)),
            pl.BlockSpec((BSQ, 2 * QW), lambda i: (i, 0)),
            pl.BlockSpec((BSQ, 2 * KW), lambda i: (i, 0)),
            pl.BlockSpec((BSQ, 2 * KW), lambda i: (i, 0)),
        ],
        out_specs=[
            pl.BlockSpec((BSQ, 2 * QW), lambda i: (i, 0)),
            pl.BlockSpec((BSQ, 2 * KW), lambda i: (i, 0)),
            pl.BlockSpec((BSQ, KV * HD), lambda i: (i, 0)),
        ],
        out_shape=[
            jax.ShapeDtypeStruct((S, 2 * QW), _F32),
            jax.ShapeDtypeStruct((S, 2 * KW), _F32),
            jax.ShapeDtypeStruct((S, KV * HD), _F32),
        ],
    )(h, ln1, wq_p, wk_p, wv, cq, sq, ck, sk)


BQ = 256
NQB = S // BQ


def _attn_body(q_ref, k_ref, v_ref, o_ref):
    qb_i = pl.program_id(0)
    q = q_ref[...]
    rows = jax.lax.broadcasted_iota(jnp.int32, (BQ, BQ), 0) + qb_i * BQ
    cols_l = jax.lax.broadcasted_iota(jnp.int32, (BQ, BQ), 1)
    scale = 1.0 / (HD ** 0.5)
    for h in range(H):
        j = h // 2
        qh = jnp.concatenate(
            [q[:, h * HH:(h + 1) * HH], q[:, QW + h * HH:QW + (h + 1) * HH]],
            axis=1)

        def step(kb, carry):
            m, l, acc = carry
            ks = jnp.concatenate(
                [k_ref[pl.ds(kb * BQ, BQ), j * HH:(j + 1) * HH],
                 k_ref[pl.ds(kb * BQ, BQ), KW + j * HH:KW + (j + 1) * HH]],
                axis=1)
            vs = v_ref[pl.ds(kb * BQ, BQ), j * HD:(j + 1) * HD]
            s = jnp.dot(qh, ks.T, preferred_element_type=_F32) * scale
            s = s + jnp.where(cols_l + kb * BQ <= rows, 0.0, -1e9)
            mnew = jnp.maximum(m, jnp.max(s, axis=-1, keepdims=True))
            p = jnp.exp(s - mnew)
            corr = jnp.exp(m - mnew)
            l = l * corr + jnp.sum(p, axis=-1, keepdims=True)
            acc = acc * corr + jnp.dot(p, vs, preferred_element_type=_F32)
            return mnew, l, acc

        init = (jnp.full((BQ, 1), -jnp.inf, _F32),
                jnp.zeros((BQ, 1), _F32),
                jnp.zeros((BQ, HD), _F32))
        m, l, acc = jax.lax.fori_loop(0, qb_i + 1, step, init)
        o_ref[:, h * HD:(h + 1) * HD] = acc / l


def _attn_call(q, k, v):
    return pl.pallas_call(
        _attn_body,
        grid=(NQB,),
        in_specs=[
            pl.BlockSpec((BQ, 2 * QW), lambda qb: (qb, 0)),
            pl.BlockSpec((S, 2 * KW), lambda qb: (0, 0)),
            pl.BlockSpec((S, KV * HD), lambda qb: (0, 0)),
        ],
        out_specs=pl.BlockSpec((BQ, H * HD), lambda qb: (qb, 0)),
        out_shape=jax.ShapeDtypeStruct((S, H * HD), _F32),
        compiler_params=pltpu.CompilerParams(
            vmem_limit_bytes=100 * 1024 * 1024),
    )(q, k, v)


def _wo_router_body(h_ref, o_ref, wo_ref, ln2_ref, wg_ref,
                    h2_out, r2_out, tw_out, eid_out, rank_out, cnt_out,
                    run_ref):
    i = pl.program_id(0)

    @pl.when(i == 0)
    def _():
        run_ref[...] = jnp.zeros((1, E), _F32)

    h2 = h_ref[...] + jnp.dot(o_ref[...], wo_ref[...],
                              preferred_element_type=_F32)
    h2_out[...] = h2
    r2 = _rms(h2, ln2_ref[...])
    r2_out[...] = r2
    logits = jnp.dot(r2, wg_ref[...], preferred_element_type=_F32)
    probs = jax.nn.softmax(logits, axis=-1)
    idx = jax.lax.broadcasted_iota(jnp.int32, (BSQ, E), 1)
    m1 = jnp.max(probs, axis=-1, keepdims=True)
    i1 = jnp.min(jnp.where(probs == m1, idx, E), axis=-1, keepdims=True)
    oh1 = (idx == i1).astype(_F32)
    rest = jnp.where(idx == i1, -jnp.inf, probs)
    m2 = jnp.max(rest, axis=-1, keepdims=True)
    i2 = jnp.min(jnp.where(rest == m2, idx, E), axis=-1, keepdims=True)
    oh2 = (idx == i2).astype(_F32)
    denom = m1 + m2
    tw_out[...] = jnp.concatenate([m1 / denom, m2 / denom], axis=1)
    eid_out[...] = jnp.concatenate([i1, i2], axis=1)

    oh12 = oh1 + oh2
    tr = jax.lax.broadcasted_iota(jnp.int32, (BSQ, BSQ), 0)
    tc = jax.lax.broadcasted_iota(jnp.int32, (BSQ, BSQ), 1)
    strict = (tc < tr).astype(_F32)
    pref = jax.lax.dot(strict, oh12, precision=_HI) + run_ref[...]
    rank1 = jnp.sum(oh1 * pref, axis=-1, keepdims=True)
    rank2 = jnp.sum(oh2 * pref, axis=-1, keepdims=True)
    rank_out[...] = jnp.concatenate([rank1, rank2], axis=1).astype(jnp.int32)
    run_ref[...] += jnp.sum(oh12, axis=0, keepdims=True)

    @pl.when(i == NSB - 1)
    def _():
        cnt_out[...] = run_ref[...].astype(jnp.int32)


def _wo_router_call(h, o, wo, ln2, wg):
    return pl.pallas_call(
        _wo_router_body,
        grid=(NSB,),
        in_specs=[
            pl.BlockSpec((BSQ, D), lambda i: (i, 0)),
            pl.BlockSpec((BSQ, H * HD), lambda i: (i, 0)),
            pl.BlockSpec((H * HD, D), lambda i: (0, 0)),
            pl.BlockSpec((D,), lambda i: (0,)),
            pl.BlockSpec((D, E), lambda i: (0, 0)),
        ],
        out_specs=[
            pl.BlockSpec((BSQ, D), lambda i: (i, 0)),
            pl.BlockSpec((BSQ, D), lambda i: (i, 0)),
            pl.BlockSpec((BSQ, TOPK), lambda i: (i, 0)),
            pl.BlockSpec((BSQ, TOPK), lambda i: (i, 0)),
            pl.BlockSpec((BSQ, TOPK), lambda i: (i, 0)),
            pl.BlockSpec((1, E), lambda i: (0, 0)),
        ],
        out_shape=[
            jax.ShapeDtypeStruct((S, D), _F32),
            jax.ShapeDtypeStruct((S, D), _F32),
            jax.ShapeDtypeStruct((S, TOPK), _F32),
            jax.ShapeDtypeStruct((S, TOPK), jnp.int32),
            jax.ShapeDtypeStruct((S, TOPK), jnp.int32),
            jax.ShapeDtypeStruct((1, E), jnp.int32),
        ],
        scratch_shapes=[pltpu.VMEM((1, E), _F32)],
    )(h, o, wo, ln2, wg)


def _fixup_body(cnt_ref, eid_ref, rank_ref,
                pos0_out, pos1_out, b_out, e_out, val_out):
    cnt = cnt_ref[...]
    pc = ((cnt + BLK - 1) // BLK) * BLK
    er = jax.lax.broadcasted_iota(jnp.int32, (E, E + 1), 0)
    jc = jax.lax.broadcasted_iota(jnp.int32, (E, E + 1), 1)
    mcum = (er < jc).astype(_F32)
    offp9 = jax.lax.dot(pc.astype(_F32), mcum,
                        precision=_HI).astype(jnp.int32)
    blkb = offp9 // BLK
    nbp = blkb[:, E:]
    g_col = jax.lax.broadcasted_iota(jnp.int32, (G, 1), 0)
    b_of_g = jnp.minimum(g_col, jnp.broadcast_to(nbp, (G, 1)) - 1)
    cmp = (jnp.broadcast_to(blkb, (G, E + 1)) <= b_of_g)
    e_of_g = jnp.clip(jnp.sum(cmp.astype(jnp.int32), axis=-1, keepdims=True)
                      - 1, 0, E - 1)
    b_out[...] = b_of_g
    e_out[...] = e_of_g
    val_out[...] = (g_col < jnp.broadcast_to(nbp, (G, 1))).astype(jnp.int32)

    te = jax.lax.broadcasted_iota(jnp.int32, (S, E), 1)
    off8b = jnp.broadcast_to(offp9[:, :E], (S, E))
    for k, out in ((0, pos0_out), (1, pos1_out)):
        ohk = te == eid_ref[:, k:k + 1]
        offsel = jnp.sum(jnp.where(ohk, off8b, 0), axis=-1, keepdims=True)
        out[...] = offsel + rank_ref[:, k:k + 1]


def _fixup_call(cnt, eid, rank):
    return pl.pallas_call(
        _fixup_body,
        out_shape=[
            jax.ShapeDtypeStruct((S, 1), jnp.int32),
            jax.ShapeDtypeStruct((S, 1), jnp.int32),
            jax.ShapeDtypeStruct((G, 1), jnp.int32),
            jax.ShapeDtypeStruct((G, 1), jnp.int32),
            jax.ShapeDtypeStruct((G, 1), jnp.int32),
        ],
    )(cnt, eid, rank)


def _make_sc_dispatch():
    from jax.experimental.pallas import tpu_sc as plsc
    per_w = S // SC_NW
    CH = 32
    nch = per_w // CH
    mesh = plsc.VectorSubcoreMesh(core_axis_name="c", subcore_axis_name="s",
                                  num_cores=SC_NC)

    @functools.partial(
        pl.kernel, mesh=mesh,
        out_type=jax.ShapeDtypeStruct((A_PAD, D), _F32),
        scratch_types=[
            pltpu.VMEM((CH,), jnp.int32),
            pltpu.VMEM((CH, D), _F32),
            pltpu.SemaphoreType.DMA,
        ],
    )
    def dispatch_k(r2_hbm, p0_hbm, p1_hbm, out_hbm, idx_v, rows_v, sem):
        wid = jax.lax.axis_index("s") * SC_NC + jax.lax.axis_index("c")
        for k in range(TOPK):
            idx_hbm = (p0_hbm, p1_hbm)[k]
            for c in range(nch):
                base = wid * per_w + c * CH
                pltpu.sync_copy(r2_hbm.at[pl.ds(base, CH)], rows_v)
                pltpu.sync_copy(idx_hbm.at[pl.ds(base, CH)], idx_v)
                pltpu.async_copy(rows_v, out_hbm.at[idx_v], sem).wait()

    return dispatch_k


def _make_sc_combine():
    from jax.experimental.pallas import tpu_sc as plsc
    per_w = S // SC_NW
    CH = 32
    nch = per_w // CH
    mesh = plsc.VectorSubcoreMesh(core_axis_name="c", subcore_axis_name="s",
                                  num_cores=SC_NC)

    @functools.partial(
        pl.kernel, mesh=mesh,
        out_type=jax.ShapeDtypeStruct((A, D), _F32),
        scratch_types=[
            pltpu.VMEM((CH,), jnp.int32),
            pltpu.VMEM((CH, D), _F32),
            pltpu.SemaphoreType.DMA,
        ],
    )
    def combine_k(y_hbm, p0_hbm, p1_hbm, out_hbm, idx_v, rows_v, sem):
        wid = jax.lax.axis_index("s") * SC_NC + jax.lax.axis_index("c")
        for k in range(TOPK):
            idx_hbm = (p0_hbm, p1_hbm)[k]
            for c in range(nch):
                base = wid * per_w + c * CH
                pltpu.sync_copy(idx_hbm.at[pl.ds(base, CH)], idx_v)
                pltpu.async_copy(y_hbm.at[idx_v], rows_v, sem).wait()
                pltpu.sync_copy(rows_v, out_hbm.at[pl.ds(k * S + base, CH)])

    return combine_k


_sc_cache = {}


def _sc_dispatch(r2, p0, p1):
    if 'd' not in _sc_cache:
        _sc_cache['d'] = _make_sc_dispatch()
    return _sc_cache['d'](r2, p0, p1)


def _sc_combine(y, p0, p1):
    if 'c' not in _sc_cache:
        _sc_cache['c'] = _make_sc_combine()
    return _sc_cache['c'](y, p0, p1)


def _gmm_body(b_arr, e_arr, val_arr,
              x_ref, w1_ref, w3_ref, w2_ref, out_ref):
    g = pl.program_id(0)

    @pl.when(val_arr[g] > 0)
    def _():
        x = x_ref[...]
        a = jnp.dot(x, w1_ref[0], preferred_element_type=_F32)
        bb = jnp.dot(x, w3_ref[0], preferred_element_type=_F32)
        out_ref[...] = jnp.dot(a * jax.nn.sigmoid(a) * bb, w2_ref[0],
                               preferred_element_type=_F32)


def _gmm_call(xg, w1, w3, w2, b_arr, e_arr, val_arr):
    grid_spec = pltpu.PrefetchScalarGridSpec(
        num_scalar_prefetch=3,
        grid=(G,),
        in_specs=[
            pl.BlockSpec((BLK, D), lambda g, bs, es, vs: (bs[g], 0)),
            pl.BlockSpec((1, D, F), lambda g, bs, es, vs: (es[g], 0, 0)),
            pl.BlockSpec((1, D, F), lambda g, bs, es, vs: (es[g], 0, 0)),
            pl.BlockSpec((1, F, D), lambda g, bs, es, vs: (es[g], 0, 0)),
        ],
        out_specs=pl.BlockSpec((BLK, D), lambda g, bs, es, vs: (bs[g], 0)),
    )
    return pl.pallas_call(
        _gmm_body,
        grid_spec=grid_spec,
        out_shape=jax.ShapeDtypeStruct((A_PAD, D), _F32),
        compiler_params=pltpu.CompilerParams(
            vmem_limit_bytes=100 * 1024 * 1024),
    )(b_arr, e_arr, val_arr, xg, w1, w3, w2)


def _combine_body(h2_ref, ya_ref, yb_ref, tw_ref, out_ref):
    tw = tw_ref[...]
    out_ref[...] = (h2_ref[...] + tw[:, 0:1] * ya_ref[...]
                    + tw[:, 1:2] * yb_ref[...])


def _combine_call(h2, yg, tw):
    return pl.pallas_call(
        _combine_body,
        grid=(NSB,),
        in_specs=[
            pl.BlockSpec((BSQ, D), lambda i: (i, 0)),
            pl.BlockSpec((BSQ, D), lambda i: (i, 0)),
            pl.BlockSpec((BSQ, D), lambda i: (S // BSQ + i, 0)),
            pl.BlockSpec((BSQ, TOPK), lambda i: (i, 0)),
        ],
        out_specs=pl.BlockSpec((BSQ, D), lambda i: (i, 0)),
        out_shape=jax.ShapeDtypeStruct((S, D), _F32),
    )(h2, yg, yg, tw)


def _final_body(h_ref, w_ref, out_ref):
    out_ref[...] = _rms(h_ref[...], w_ref[...])


def _final_call(h, w):
    return pl.pallas_call(
        _final_body,
        out_shape=jax.ShapeDtypeStruct((S, D), _F32),
    )(h, w)


def _col_perm_q():
    import numpy as np
    n = np.arange(2 * QW)
    half, rest = n // QW, n % QW
    return (rest // HH) * HD + half * HH + rest % HH


def _col_perm_k():
    import numpy as np
    n = np.arange(2 * KW)
    half, rest = n // KW, n % KW
    return (rest // HH) * HD + half * HH + rest % HH


def _rope_tables():
    inv_freq = 1.0 / (THETA ** (jnp.arange(0, HD, 2).astype(_F32) / HD))
    freqs = jnp.arange(S, dtype=_F32)[:, None] * inv_freq[None, :]
    cosf, sinf = jnp.cos(freqs), jnp.sin(freqs)
    cq = jnp.tile(cosf, (1, 2 * QW // HH))
    sq = jnp.tile(sinf, (1, 2 * QW // HH))
    ck = jnp.tile(cosf, (1, 2 * KW // HH))
    sk = jnp.tile(sinf, (1, 2 * KW // HH))
    return cq, sq, ck, sk


@jax.jit
def _forward(x, params):
    cq, sq, ck, sk = _rope_tables()
    pq, pk = _col_perm_q(), _col_perm_k()
    h = x.reshape(S, D)
    for l in range(L):
        p = params['layer_%d' % l]
        q, k, v = _qkv_call(h, p['ln1'], p['wq'][:, pq], p['wk'][:, pk],
                            p['wv'], cq, sq, ck, sk)
        o = _attn_call(q, k, v)
        h2, r2, tw, eid, rank, cnt = _wo_router_call(
            h, o, p['wo'], p['ln2'], p['wg'])
        pos0, pos1, b_arr, e_arr, val_arr = _fixup_call(cnt, eid, rank)
        p0 = pos0.reshape(S)
        p1 = pos1.reshape(S)
        xg = _sc_dispatch(r2, p0, p1)
        y = _gmm_call(xg, p['w1'], p['w3'], p['w2'],
                      b_arr.reshape(G), e_arr.reshape(G),
                      val_arr.reshape(G))
        yg = _sc_combine(y, p0, p1)
        h = _combine_call(h2, yg, tw)
    return _final_call(h, params['final_ln']).reshape(1, S, D)


def kernel(input_ids, params):
    return _forward(input_ids, params)


# speedup vs baseline: 1.4930x; 1.4930x over previous
import functools

import jax
import jax.numpy as jnp
from jax.experimental import pallas as pl
from jax.experimental.pallas import tpu as pltpu

S, D = 2048, 1024
H, KV, HD = 16, 8, 64
E, TOPK, F = 8, 2, 1024
L = 2
EPS = 1e-6
THETA = 10000.0
HH = HD // 2

QW = H * HH
KW = KV * HH

BSQ = 512
NSB = S // BSQ

A = TOPK * S
BLK = 256
NB = A // BLK
G = NB + E - 1
A_PAD = G * BLK

SC_NC, SC_NS = 2, 16
SC_NW = SC_NC * SC_NS

_F32 = jnp.float32
_HI = jax.lax.Precision.HIGHEST


def _rms(x, w):
    return x * jax.lax.rsqrt(jnp.mean(x * x, axis=-1, keepdims=True) + EPS) * w


def _qkv_body(h_ref, ln1_ref, wq_ref, wk_ref, wv_ref, cq_ref, sq_ref,
              ck_ref, sk_ref, q_out, k_out, v_out):
    r = _rms(h_ref[...], ln1_ref[...])
    q = jnp.dot(r, wq_ref[...], preferred_element_type=_F32)
    k = jnp.dot(r, wk_ref[...], preferred_element_type=_F32)
    v = jnp.dot(r, wv_ref[...], preferred_element_type=_F32)
    qr = jnp.concatenate([-q[:, QW:], q[:, :QW]], axis=1)
    kr = jnp.concatenate([-k[:, KW:], k[:, :KW]], axis=1)
    q_out[...] = q * cq_ref[...] + qr * sq_ref[...]
    k_out[...] = k * ck_ref[...] + kr * sk_ref[...]
    v_out[...] = v


def _qkv_call(h, ln1, wq_p, wk_p, wv, cq, sq, ck, sk):
    return pl.pallas_call(
        _qkv_body,
        grid=(NSB,),
        in_specs=[
            pl.BlockSpec((BSQ, D), lambda i: (i, 0)),
            pl.BlockSpec((D,), lambda i: (0,)),
            pl.BlockSpec((D, 2 * QW), lambda i: (0, 0)),
            pl.BlockSpec((D, 2 * KW), lambda i: (0, 0)),
            pl.BlockSpec((D, KV * HD), lambda i: (0, 0)),
            pl.BlockSpec((BSQ, 2 * QW), lambda i: (i, 0)),
            pl.BlockSpec((BSQ, 2 * QW), lambda i: (i, 0)),
            pl.BlockSpec((BSQ, 2 * KW), lambda i: (i, 0)),
            pl.BlockSpec((BSQ, 2 * KW), lambda i: (i, 0)),
        ],
        out_specs=[
            pl.BlockSpec((BSQ, 2 * QW), lambda i: (i, 0)),
            pl.BlockSpec((BSQ, 2 * KW), lambda i: (i, 0)),
            pl.BlockSpec((BSQ, KV * HD), lambda i: (i, 0)),
        ],
        out_shape=[
            jax.ShapeDtypeStruct((S, 2 * QW), _F32),
            jax.ShapeDtypeStruct((S, 2 * KW), _F32),
            jax.ShapeDtypeStruct((S, KV * HD), _F32),
        ],
    )(h, ln1, wq_p, wk_p, wv, cq, sq, ck, sk)


BQ = 512
NQB = S // BQ


def _attn_body(q_ref, k_ref, v_ref, o_ref):
    qb_i = pl.program_id(0)
    q = q_ref[...]
    k = k_ref[...]
    v = v_ref[...]
    rows = jax.lax.broadcasted_iota(jnp.int32, (BQ, S), 0) + qb_i * BQ
    cols = jax.lax.broadcasted_iota(jnp.int32, (BQ, S), 1)
    bias = jnp.where(cols <= rows, 0.0, -1e9)
    for h in range(H):
        j = h // 2
        qh = jnp.concatenate(
            [q[:, h * HH:(h + 1) * HH], q[:, QW + h * HH:QW + (h + 1) * HH]],
            axis=1)
        kh = jnp.concatenate(
            [k[:, j * HH:(j + 1) * HH], k[:, KW + j * HH:KW + (j + 1) * HH]],
            axis=1)
        s = jnp.dot(qh, kh.T, preferred_element_type=_F32)
        s = s * (1.0 / (HD ** 0.5)) + bias
        p = jax.nn.softmax(s, axis=-1)
        o_ref[:, h * HD:(h + 1) * HD] = jnp.dot(
            p, v[:, j * HD:(j + 1) * HD], preferred_element_type=_F32)


def _attn_call(q, k, v):
    return pl.pallas_call(
        _attn_body,
        grid=(NQB,),
        in_specs=[
            pl.BlockSpec((BQ, 2 * QW), lambda qb: (qb, 0)),
            pl.BlockSpec((S, 2 * KW), lambda qb: (0, 0)),
            pl.BlockSpec((S, KV * HD), lambda qb: (0, 0)),
        ],
        out_specs=pl.BlockSpec((BQ, H * HD), lambda qb: (qb, 0)),
        out_shape=jax.ShapeDtypeStruct((S, H * HD), _F32),
        compiler_params=pltpu.CompilerParams(
            vmem_limit_bytes=100 * 1024 * 1024),
    )(q, k, v)


def _wo_router_body(h_ref, o_ref, wo_ref, ln2_ref, wg_ref,
                    h2_out, r2_out, tw_out, eid_out, rank_out, cnt_out,
                    run_ref):
    i = pl.program_id(0)

    @pl.when(i == 0)
    def _():
        run_ref[...] = jnp.zeros((1, E), _F32)

    h2 = h_ref[...] + jnp.dot(o_ref[...], wo_ref[...],
                              preferred_element_type=_F32)
    h2_out[...] = h2
    r2 = _rms(h2, ln2_ref[...])
    r2_out[...] = r2
    logits = jnp.dot(r2, wg_ref[...], preferred_element_type=_F32)
    probs = jax.nn.softmax(logits, axis=-1)
    idx = jax.lax.broadcasted_iota(jnp.int32, (BSQ, E), 1)
    m1 = jnp.max(probs, axis=-1, keepdims=True)
    i1 = jnp.min(jnp.where(probs == m1, idx, E), axis=-1, keepdims=True)
    oh1 = (idx == i1).astype(_F32)
    rest = jnp.where(idx == i1, -jnp.inf, probs)
    m2 = jnp.max(rest, axis=-1, keepdims=True)
    i2 = jnp.min(jnp.where(rest == m2, idx, E), axis=-1, keepdims=True)
    oh2 = (idx == i2).astype(_F32)
    denom = m1 + m2
    tw_out[...] = jnp.concatenate([m1 / denom, m2 / denom], axis=1)
    eid_out[...] = jnp.concatenate([i1, i2], axis=1)

    oh12 = oh1 + oh2
    tr = jax.lax.broadcasted_iota(jnp.int32, (BSQ, BSQ), 0)
    tc = jax.lax.broadcasted_iota(jnp.int32, (BSQ, BSQ), 1)
    strict = (tc < tr).astype(_F32)
    pref = jax.lax.dot(strict, oh12, precision=_HI) + run_ref[...]
    rank1 = jnp.sum(oh1 * pref, axis=-1, keepdims=True)
    rank2 = jnp.sum(oh2 * pref, axis=-1, keepdims=True)
    rank_out[...] = jnp.concatenate([rank1, rank2], axis=1).astype(jnp.int32)
    run_ref[...] += jnp.sum(oh12, axis=0, keepdims=True)

    @pl.when(i == NSB - 1)
    def _():
        cnt_out[...] = run_ref[...].astype(jnp.int32)


def _wo_router_call(h, o, wo, ln2, wg):
    return pl.pallas_call(
        _wo_router_body,
        grid=(NSB,),
        in_specs=[
            pl.BlockSpec((BSQ, D), lambda i: (i, 0)),
            pl.BlockSpec((BSQ, H * HD), lambda i: (i, 0)),
            pl.BlockSpec((H * HD, D), lambda i: (0, 0)),
            pl.BlockSpec((D,), lambda i: (0,)),
            pl.BlockSpec((D, E), lambda i: (0, 0)),
        ],
        out_specs=[
            pl.BlockSpec((BSQ, D), lambda i: (i, 0)),
            pl.BlockSpec((BSQ, D), lambda i: (i, 0)),
            pl.BlockSpec((BSQ, TOPK), lambda i: (i, 0)),
            pl.BlockSpec((BSQ, TOPK), lambda i: (i, 0)),
            pl.BlockSpec((BSQ, TOPK), lambda i: (i, 0)),
            pl.BlockSpec((1, E), lambda i: (0, 0)),
        ],
        out_shape=[
            jax.ShapeDtypeStruct((S, D), _F32),
            jax.ShapeDtypeStruct((S, D), _F32),
            jax.ShapeDtypeStruct((S, TOPK), _F32),
            jax.ShapeDtypeStruct((S, TOPK), jnp.int32),
            jax.ShapeDtypeStruct((S, TOPK), jnp.int32),
            jax.ShapeDtypeStruct((1, E), jnp.int32),
        ],
        scratch_shapes=[pltpu.VMEM((1, E), _F32)],
    )(h, o, wo, ln2, wg)


def _fixup_body(cnt_ref, eid_ref, rank_ref,
                pos0_out, pos1_out, b_out, e_out, val_out):
    cnt = cnt_ref[...]
    pc = ((cnt + BLK - 1) // BLK) * BLK
    er = jax.lax.broadcasted_iota(jnp.int32, (E, E + 1), 0)
    jc = jax.lax.broadcasted_iota(jnp.int32, (E, E + 1), 1)
    mcum = (er < jc).astype(_F32)
    offp9 = jax.lax.dot(pc.astype(_F32), mcum,
                        precision=_HI).astype(jnp.int32)
    blkb = offp9 // BLK
    nbp = blkb[:, E:]
    g_col = jax.lax.broadcasted_iota(jnp.int32, (G, 1), 0)
    b_of_g = jnp.minimum(g_col, jnp.broadcast_to(nbp, (G, 1)) - 1)
    cmp = (jnp.broadcast_to(blkb, (G, E + 1)) <= b_of_g)
    e_of_g = jnp.clip(jnp.sum(cmp.astype(jnp.int32), axis=-1, keepdims=True)
                      - 1, 0, E - 1)
    b_out[...] = b_of_g
    e_out[...] = e_of_g
    val_out[...] = (g_col < jnp.broadcast_to(nbp, (G, 1))).astype(jnp.int32)

    te = jax.lax.broadcasted_iota(jnp.int32, (S, E), 1)
    off8b = jnp.broadcast_to(offp9[:, :E], (S, E))
    for k, out in ((0, pos0_out), (1, pos1_out)):
        ohk = te == eid_ref[:, k:k + 1]
        offsel = jnp.sum(jnp.where(ohk, off8b, 0), axis=-1, keepdims=True)
        out[...] = offsel + rank_ref[:, k:k + 1]


def _fixup_call(cnt, eid, rank):
    return pl.pallas_call(
        _fixup_body,
        out_shape=[
            jax.ShapeDtypeStruct((S, 1), jnp.int32),
            jax.ShapeDtypeStruct((S, 1), jnp.int32),
            jax.ShapeDtypeStruct((G, 1), jnp.int32),
            jax.ShapeDtypeStruct((G, 1), jnp.int32),
            jax.ShapeDtypeStruct((G, 1), jnp.int32),
        ],
    )(cnt, eid, rank)


def _make_sc_dispatch():
    from jax.experimental.pallas import tpu_sc as plsc
    per_w = S // SC_NW
    CH = 32
    nch = per_w // CH
    mesh = plsc.VectorSubcoreMesh(core_axis_name="c", subcore_axis_name="s",
                                  num_cores=SC_NC)

    @functools.partial(
        pl.kernel, mesh=mesh,
        out_type=jax.ShapeDtypeStruct((A_PAD, D), _F32),
        scratch_types=[
            pltpu.VMEM((CH,), jnp.int32),
            pltpu.VMEM((CH, D), _F32),
            pltpu.SemaphoreType.DMA,
        ],
    )
    def dispatch_k(r2_hbm, p0_hbm, p1_hbm, out_hbm, idx_v, rows_v, sem):
        wid = jax.lax.axis_index("s") * SC_NC + jax.lax.axis_index("c")
        for k in range(TOPK):
            idx_hbm = (p0_hbm, p1_hbm)[k]
            for c in range(nch):
                base = wid * per_w + c * CH
                pltpu.sync_copy(r2_hbm.at[pl.ds(base, CH)], rows_v)
                pltpu.sync_copy(idx_hbm.at[pl.ds(base, CH)], idx_v)
                pltpu.async_copy(rows_v, out_hbm.at[idx_v], sem).wait()

    return dispatch_k


def _make_sc_combine():
    from jax.experimental.pallas import tpu_sc as plsc
    per_w = S // SC_NW
    CH = 32
    nch = per_w // CH
    mesh = plsc.VectorSubcoreMesh(core_axis_name="c", subcore_axis_name="s",
                                  num_cores=SC_NC)

    @functools.partial(
        pl.kernel, mesh=mesh,
        out_type=jax.ShapeDtypeStruct((A, D), _F32),
        scratch_types=[
            pltpu.VMEM((CH,), jnp.int32),
            pltpu.VMEM((CH, D), _F32),
            pltpu.SemaphoreType.DMA,
        ],
    )
    def combine_k(y_hbm, p0_hbm, p1_hbm, out_hbm, idx_v, rows_v, sem):
        wid = jax.lax.axis_index("s") * SC_NC + jax.lax.axis_index("c")
        for k in range(TOPK):
            idx_hbm = (p0_hbm, p1_hbm)[k]
            for c in range(nch):
                base = wid * per_w + c * CH
                pltpu.sync_copy(idx_hbm.at[pl.ds(base, CH)], idx_v)
                pltpu.async_copy(y_hbm.at[idx_v], rows_v, sem).wait()
                pltpu.sync_copy(rows_v, out_hbm.at[pl.ds(k * S + base, CH)])

    return combine_k


_sc_cache = {}


def _sc_dispatch(r2, p0, p1):
    if 'd' not in _sc_cache:
        _sc_cache['d'] = _make_sc_dispatch()
    return _sc_cache['d'](r2, p0, p1)


def _sc_combine(y, p0, p1):
    if 'c' not in _sc_cache:
        _sc_cache['c'] = _make_sc_combine()
    return _sc_cache['c'](y, p0, p1)


def _gmm_body(b_arr, e_arr, val_arr,
              x_ref, w1_ref, w3_ref, w2_ref, out_ref):
    g = pl.program_id(0)

    @pl.when(val_arr[g] > 0)
    def _():
        x = x_ref[...]
        a = jnp.dot(x, w1_ref[0], preferred_element_type=_F32)
        bb = jnp.dot(x, w3_ref[0], preferred_element_type=_F32)
        out_ref[...] = jnp.dot(a * jax.nn.sigmoid(a) * bb, w2_ref[0],
                               preferred_element_type=_F32)


def _gmm_call(xg, w1, w3, w2, b_arr, e_arr, val_arr):
    grid_spec = pltpu.PrefetchScalarGridSpec(
        num_scalar_prefetch=3,
        grid=(G,),
        in_specs=[
            pl.BlockSpec((BLK, D), lambda g, bs, es, vs: (bs[g], 0)),
            pl.BlockSpec((1, D, F), lambda g, bs, es, vs: (es[g], 0, 0)),
            pl.BlockSpec((1, D, F), lambda g, bs, es, vs: (es[g], 0, 0)),
            pl.BlockSpec((1, F, D), lambda g, bs, es, vs: (es[g], 0, 0)),
        ],
        out_specs=pl.BlockSpec((BLK, D), lambda g, bs, es, vs: (bs[g], 0)),
    )
    return pl.pallas_call(
        _gmm_body,
        grid_spec=grid_spec,
        out_shape=jax.ShapeDtypeStruct((A_PAD, D), _F32),
        compiler_params=pltpu.CompilerParams(
            vmem_limit_bytes=100 * 1024 * 1024),
    )(b_arr, e_arr, val_arr, xg, w1, w3, w2)


def _combine_body(h2_ref, ya_ref, yb_ref, tw_ref, out_ref):
    tw = tw_ref[...]
    out_ref[...] = (h2_ref[...] + tw[:, 0:1] * ya_ref[...]
                    + tw[:, 1:2] * yb_ref[...])


def _combine_call(h2, yg, tw):
    return pl.pallas_call(
        _combine_body,
        grid=(NSB,),
        in_specs=[
            pl.BlockSpec((BSQ, D), lambda i: (i, 0)),
            pl.BlockSpec((BSQ, D), lambda i: (i, 0)),
            pl.BlockSpec((BSQ, D), lambda i: (S // BSQ + i, 0)),
            pl.BlockSpec((BSQ, TOPK), lambda i: (i, 0)),
        ],
        out_specs=pl.BlockSpec((BSQ, D), lambda i: (i, 0)),
        out_shape=jax.ShapeDtypeStruct((S, D), _F32),
    )(h2, yg, yg, tw)


def _final_body(h_ref, w_ref, out_ref):
    out_ref[...] = _rms(h_ref[...], w_ref[...])


def _final_call(h, w):
    return pl.pallas_call(
        _final_body,
        out_shape=jax.ShapeDtypeStruct((S, D), _F32),
    )(h, w)


def _col_perm_q():
    import numpy as np
    n = np.arange(2 * QW)
    half, rest = n // QW, n % QW
    return (rest // HH) * HD + half * HH + rest % HH


def _col_perm_k():
    import numpy as np
    n = np.arange(2 * KW)
    half, rest = n // KW, n % KW
    return (rest // HH) * HD + half * HH + rest % HH


def _rope_tables():
    inv_freq = 1.0 / (THETA ** (jnp.arange(0, HD, 2).astype(_F32) / HD))
    freqs = jnp.arange(S, dtype=_F32)[:, None] * inv_freq[None, :]
    cosf, sinf = jnp.cos(freqs), jnp.sin(freqs)
    cq = jnp.tile(cosf, (1, 2 * QW // HH))
    sq = jnp.tile(sinf, (1, 2 * QW // HH))
    ck = jnp.tile(cosf, (1, 2 * KW // HH))
    sk = jnp.tile(sinf, (1, 2 * KW // HH))
    return cq, sq, ck, sk


@jax.jit
def _forward(x, params):
    cq, sq, ck, sk = _rope_tables()
    pq, pk = _col_perm_q(), _col_perm_k()
    h = x.reshape(S, D)
    for l in range(L):
        p = params['layer_%d' % l]
        q, k, v = _qkv_call(h, p['ln1'], p['wq'][:, pq], p['wk'][:, pk],
                            p['wv'], cq, sq, ck, sk)
        o = _attn_call(q, k, v)
        h2, r2, tw, eid, rank, cnt = _wo_router_call(
            h, o, p['wo'], p['ln2'], p['wg'])
        pos0, pos1, b_arr, e_arr, val_arr = _fixup_call(cnt, eid, rank)
        p0 = pos0.reshape(S)
        p1 = pos1.reshape(S)
        xg = _sc_dispatch(r2, p0, p1)
        y = _gmm_call(xg, p['w1'], p['w3'], p['w2'],
                      b_arr.reshape(G), e_arr.reshape(G),
                      val_arr.reshape(G))
        yg = _sc_combine(y, p0, p1)
        h = _combine_call(h2, yg, tw)
    return _final_call(h, params['final_ln']).reshape(1, S, D)


def kernel(input_ids, params):
    return _forward(input_ids, params)


# speedup vs baseline: 1.6231x; 1.0872x over previous
import functools

import jax
import jax.numpy as jnp
from jax.experimental import pallas as pl
from jax.experimental.pallas import tpu as pltpu

S, D = 2048, 1024
H, KV, HD = 16, 8, 64
E, TOPK, F = 8, 2, 1024
L = 2
EPS = 1e-6
THETA = 10000.0
HH = HD // 2

QW = H * HH
KW = KV * HH

BSQ = 512
NSB = S // BSQ

A = TOPK * S
BLK = 256
NB = A // BLK
G = NB + E - 1
A_PAD = G * BLK

SC_NC, SC_NS = 2, 16
SC_NW = SC_NC * SC_NS

_F32 = jnp.float32
_HI = jax.lax.Precision.HIGHEST


def _rms(x, w):
    return x * jax.lax.rsqrt(jnp.mean(x * x, axis=-1, keepdims=True) + EPS) * w


def _qkv_body(h_ref, ln1_ref, wq_ref, wk_ref, wv_ref, cq_ref, sq_ref,
              ck_ref, sk_ref, q_out, k_out, v_out):
    r = _rms(h_ref[...], ln1_ref[...])
    q = jnp.dot(r, wq_ref[...], preferred_element_type=_F32)
    k = jnp.dot(r, wk_ref[...], preferred_element_type=_F32)
    v = jnp.dot(r, wv_ref[...], preferred_element_type=_F32)
    qr = jnp.concatenate([-q[:, QW:], q[:, :QW]], axis=1)
    kr = jnp.concatenate([-k[:, KW:], k[:, :KW]], axis=1)
    q_out[...] = q * cq_ref[...] + qr * sq_ref[...]
    k_out[...] = k * ck_ref[...] + kr * sk_ref[...]
    v_out[...] = v


def _qkv_call(h, ln1, wq_p, wk_p, wv, cq, sq, ck, sk):
    return pl.pallas_call(
        _qkv_body,
        grid=(NSB,),
        in_specs=[
            pl.BlockSpec((BSQ, D), lambda i: (i, 0)),
            pl.BlockSpec((D,), lambda i: (0,)),
            pl.BlockSpec((D, 2 * QW), lambda i: (0, 0)),
            pl.BlockSpec((D, 2 * KW), lambda i: (0, 0)),
            pl.BlockSpec((D, KV * HD), lambda i: (0, 0)),
            pl.BlockSpec((BSQ, 2 * QW), lambda i: (i, 0)),
            pl.BlockSpec((BSQ, 2 * QW), lambda i: (i, 0)),
            pl.BlockSpec((BSQ, 2 * KW), lambda i: (i, 0)),
            pl.BlockSpec((BSQ, 2 * KW), lambda i: (i, 0)),
        ],
        out_specs=[
            pl.BlockSpec((BSQ, 2 * QW), lambda i: (i, 0)),
            pl.BlockSpec((BSQ, 2 * KW), lambda i: (i, 0)),
            pl.BlockSpec((BSQ, KV * HD), lambda i: (i, 0)),
        ],
        out_shape=[
            jax.ShapeDtypeStruct((S, 2 * QW), _F32),
            jax.ShapeDtypeStruct((S, 2 * KW), _F32),
            jax.ShapeDtypeStruct((S, KV * HD), _F32),
        ],
    )(h, ln1, wq_p, wk_p, wv, cq, sq, ck, sk)


BQ = 256
NQB = S // BQ


def _make_attn_body(qb0, kwid):
    def body(q_ref, k_ref, v_ref, o_ref):
        qb_i = pl.program_id(0)
        q = q_ref[...]
        k = k_ref[...]
        v = v_ref[...]
        rows = (jax.lax.broadcasted_iota(jnp.int32, (BQ, kwid), 0)
                + (qb0 + qb_i) * BQ)
        cols = jax.lax.broadcasted_iota(jnp.int32, (BQ, kwid), 1)
        bias = jnp.where(cols <= rows, 0.0, -1e9)
        for h in range(H):
            j = h // 2
            qh = jnp.concatenate(
                [q[:, h * HH:(h + 1) * HH],
                 q[:, QW + h * HH:QW + (h + 1) * HH]], axis=1)
            kh = jnp.concatenate(
                [k[:, j * HH:(j + 1) * HH],
                 k[:, KW + j * HH:KW + (j + 1) * HH]], axis=1)
            s = jnp.dot(qh, kh.T, preferred_element_type=_F32)
            s = s * (1.0 / (HD ** 0.5)) + bias
            p = jax.nn.softmax(s, axis=-1)
            o_ref[:, h * HD:(h + 1) * HD] = jnp.dot(
                p, v[:, j * HD:(j + 1) * HD], preferred_element_type=_F32)
    return body


QG = 2


def _attn_call(q, k, v):
    outs = []
    for g in range(NQB // QG):
        qb0 = g * QG
        kwid = (qb0 + QG) * BQ
        o_g = pl.pallas_call(
            _make_attn_body(qb0, kwid),
            grid=(QG,),
            in_specs=[
                pl.BlockSpec((BQ, 2 * QW), lambda qb, qb0=qb0: (qb0 + qb, 0)),
                pl.BlockSpec((kwid, 2 * KW), lambda qb: (0, 0)),
                pl.BlockSpec((kwid, KV * HD), lambda qb: (0, 0)),
            ],
            out_specs=pl.BlockSpec((BQ, H * HD), lambda qb: (qb, 0)),
            out_shape=jax.ShapeDtypeStruct((QG * BQ, H * HD), _F32),
            compiler_params=pltpu.CompilerParams(
                vmem_limit_bytes=100 * 1024 * 1024),
        )(q, k, v)
        outs.append(o_g)
    return jnp.concatenate(outs, axis=0)


def _wo_router_body(h_ref, o_ref, wo_ref, ln2_ref, wg_ref,
                    h2_out, r2_out, tw_out, eid_out, rank_out, cnt_out,
                    run_ref):
    i = pl.program_id(0)

    @pl.when(i == 0)
    def _():
        run_ref[...] = jnp.zeros((1, E), _F32)

    h2 = h_ref[...] + jnp.dot(o_ref[...], wo_ref[...],
                              preferred_element_type=_F32)
    h2_out[...] = h2
    r2 = _rms(h2, ln2_ref[...])
    r2_out[...] = r2
    logits = jnp.dot(r2, wg_ref[...], preferred_element_type=_F32)
    probs = jax.nn.softmax(logits, axis=-1)
    idx = jax.lax.broadcasted_iota(jnp.int32, (BSQ, E), 1)
    m1 = jnp.max(probs, axis=-1, keepdims=True)
    i1 = jnp.min(jnp.where(probs == m1, idx, E), axis=-1, keepdims=True)
    oh1 = (idx == i1).astype(_F32)
    rest = jnp.where(idx == i1, -jnp.inf, probs)
    m2 = jnp.max(rest, axis=-1, keepdims=True)
    i2 = jnp.min(jnp.where(rest == m2, idx, E), axis=-1, keepdims=True)
    oh2 = (idx == i2).astype(_F32)
    denom = m1 + m2
    tw_out[...] = jnp.concatenate([m1 / denom, m2 / denom], axis=1)
    eid_out[...] = jnp.concatenate([i1, i2], axis=1)

    oh12 = oh1 + oh2
    tr = jax.lax.broadcasted_iota(jnp.int32, (BSQ, BSQ), 0)
    tc = jax.lax.broadcasted_iota(jnp.int32, (BSQ, BSQ), 1)
    strict = (tc < tr).astype(_F32)
    pref = jax.lax.dot(strict, oh12, precision=_HI) + run_ref[...]
    rank1 = jnp.sum(oh1 * pref, axis=-1, keepdims=True)
    rank2 = jnp.sum(oh2 * pref, axis=-1, keepdims=True)
    rank_out[...] = jnp.concatenate([rank1, rank2], axis=1).astype(jnp.int32)
    run_ref[...] += jnp.sum(oh12, axis=0, keepdims=True)

    @pl.when(i == NSB - 1)
    def _():
        cnt_out[...] = run_ref[...].astype(jnp.int32)


def _wo_router_call(h, o, wo, ln2, wg):
    return pl.pallas_call(
        _wo_router_body,
        grid=(NSB,),
        in_specs=[
            pl.BlockSpec((BSQ, D), lambda i: (i, 0)),
            pl.BlockSpec((BSQ, H * HD), lambda i: (i, 0)),
            pl.BlockSpec((H * HD, D), lambda i: (0, 0)),
            pl.BlockSpec((D,), lambda i: (0,)),
            pl.BlockSpec((D, E), lambda i: (0, 0)),
        ],
        out_specs=[
            pl.BlockSpec((BSQ, D), lambda i: (i, 0)),
            pl.BlockSpec((BSQ, D), lambda i: (i, 0)),
            pl.BlockSpec((BSQ, TOPK), lambda i: (i, 0)),
            pl.BlockSpec((BSQ, TOPK), lambda i: (i, 0)),
            pl.BlockSpec((BSQ, TOPK), lambda i: (i, 0)),
            pl.BlockSpec((1, E), lambda i: (0, 0)),
        ],
        out_shape=[
            jax.ShapeDtypeStruct((S, D), _F32),
            jax.ShapeDtypeStruct((S, D), _F32),
            jax.ShapeDtypeStruct((S, TOPK), _F32),
            jax.ShapeDtypeStruct((S, TOPK), jnp.int32),
            jax.ShapeDtypeStruct((S, TOPK), jnp.int32),
            jax.ShapeDtypeStruct((1, E), jnp.int32),
        ],
        scratch_shapes=[pltpu.VMEM((1, E), _F32)],
    )(h, o, wo, ln2, wg)


def _fixup_body(cnt_ref, eid_ref, rank_ref,
                pos0_out, pos1_out, b_out, e_out, val_out):
    cnt = cnt_ref[...]
    pc = ((cnt + BLK - 1) // BLK) * BLK
    er = jax.lax.broadcasted_iota(jnp.int32, (E, E + 1), 0)
    jc = jax.lax.broadcasted_iota(jnp.int32, (E, E + 1), 1)
    mcum = (er < jc).astype(_F32)
    offp9 = jax.lax.dot(pc.astype(_F32), mcum,
                        precision=_HI).astype(jnp.int32)
    blkb = offp9 // BLK
    nbp = blkb[:, E:]
    g_col = jax.lax.broadcasted_iota(jnp.int32, (G, 1), 0)
    b_of_g = jnp.minimum(g_col, jnp.broadcast_to(nbp, (G, 1)) - 1)
    cmp = (jnp.broadcast_to(blkb, (G, E + 1)) <= b_of_g)
    e_of_g = jnp.clip(jnp.sum(cmp.astype(jnp.int32), axis=-1, keepdims=True)
                      - 1, 0, E - 1)
    b_out[...] = b_of_g
    e_out[...] = e_of_g
    val_out[...] = (g_col < jnp.broadcast_to(nbp, (G, 1))).astype(jnp.int32)

    te = jax.lax.broadcasted_iota(jnp.int32, (S, E), 1)
    off8b = jnp.broadcast_to(offp9[:, :E], (S, E))
    for k, out in ((0, pos0_out), (1, pos1_out)):
        ohk = te == eid_ref[:, k:k + 1]
        offsel = jnp.sum(jnp.where(ohk, off8b, 0), axis=-1, keepdims=True)
        out[...] = offsel + rank_ref[:, k:k + 1]


def _fixup_call(cnt, eid, rank):
    return pl.pallas_call(
        _fixup_body,
        out_shape=[
            jax.ShapeDtypeStruct((S, 1), jnp.int32),
            jax.ShapeDtypeStruct((S, 1), jnp.int32),
            jax.ShapeDtypeStruct((G, 1), jnp.int32),
            jax.ShapeDtypeStruct((G, 1), jnp.int32),
            jax.ShapeDtypeStruct((G, 1), jnp.int32),
        ],
    )(cnt, eid, rank)


def _make_sc_dispatch():
    from jax.experimental.pallas import tpu_sc as plsc
    per_w = S // SC_NW
    CH = 32
    nch = per_w // CH
    mesh = plsc.VectorSubcoreMesh(core_axis_name="c", subcore_axis_name="s",
                                  num_cores=SC_NC)

    @functools.partial(
        pl.kernel, mesh=mesh,
        out_type=jax.ShapeDtypeStruct((A_PAD, D), _F32),
        scratch_types=[
            pltpu.VMEM((CH,), jnp.int32),
            pltpu.VMEM((CH, D), _F32),
            pltpu.SemaphoreType.DMA,
        ],
    )
    def dispatch_k(r2_hbm, p0_hbm, p1_hbm, out_hbm, idx_v, rows_v, sem):
        wid = jax.lax.axis_index("s") * SC_NC + jax.lax.axis_index("c")
        for k in range(TOPK):
            idx_hbm = (p0_hbm, p1_hbm)[k]
            for c in range(nch):
                base = wid * per_w + c * CH
                pltpu.sync_copy(r2_hbm.at[pl.ds(base, CH)], rows_v)
                pltpu.sync_copy(idx_hbm.at[pl.ds(base, CH)], idx_v)
                pltpu.async_copy(rows_v, out_hbm.at[idx_v], sem).wait()

    return dispatch_k


def _make_sc_combine():
    from jax.experimental.pallas import tpu_sc as plsc
    per_w = S // SC_NW
    CH = 32
    nch = per_w // CH
    mesh = plsc.VectorSubcoreMesh(core_axis_name="c", subcore_axis_name="s",
                                  num_cores=SC_NC)

    @functools.partial(
        pl.kernel, mesh=mesh,
        out_type=jax.ShapeDtypeStruct((A, D), _F32),
        scratch_types=[
            pltpu.VMEM((CH,), jnp.int32),
            pltpu.VMEM((CH, D), _F32),
            pltpu.SemaphoreType.DMA,
        ],
    )
    def combine_k(y_hbm, p0_hbm, p1_hbm, out_hbm, idx_v, rows_v, sem):
        wid = jax.lax.axis_index("s") * SC_NC + jax.lax.axis_index("c")
        for k in range(TOPK):
            idx_hbm = (p0_hbm, p1_hbm)[k]
            for c in range(nch):
                base = wid * per_w + c * CH
                pltpu.sync_copy(idx_hbm.at[pl.ds(base, CH)], idx_v)
                pltpu.async_copy(y_hbm.at[idx_v], rows_v, sem).wait()
                pltpu.sync_copy(rows_v, out_hbm.at[pl.ds(k * S + base, CH)])

    return combine_k


_sc_cache = {}


def _sc_dispatch(r2, p0, p1):
    if 'd' not in _sc_cache:
        _sc_cache['d'] = _make_sc_dispatch()
    return _sc_cache['d'](r2, p0, p1)


def _sc_combine(y, p0, p1):
    if 'c' not in _sc_cache:
        _sc_cache['c'] = _make_sc_combine()
    return _sc_cache['c'](y, p0, p1)


def _gmm_body(b_arr, e_arr, val_arr,
              x_ref, w1_ref, w3_ref, w2_ref, out_ref):
    g = pl.program_id(0)

    @pl.when(val_arr[g] > 0)
    def _():
        x = x_ref[...]
        a = jnp.dot(x, w1_ref[0], preferred_element_type=_F32)
        bb = jnp.dot(x, w3_ref[0], preferred_element_type=_F32)
        out_ref[...] = jnp.dot(a * jax.nn.sigmoid(a) * bb, w2_ref[0],
                               preferred_element_type=_F32)


def _gmm_call(xg, w1, w3, w2, b_arr, e_arr, val_arr):
    grid_spec = pltpu.PrefetchScalarGridSpec(
        num_scalar_prefetch=3,
        grid=(G,),
        in_specs=[
            pl.BlockSpec((BLK, D), lambda g, bs, es, vs: (bs[g], 0)),
            pl.BlockSpec((1, D, F), lambda g, bs, es, vs: (es[g], 0, 0)),
            pl.BlockSpec((1, D, F), lambda g, bs, es, vs: (es[g], 0, 0)),
            pl.BlockSpec((1, F, D), lambda g, bs, es, vs: (es[g], 0, 0)),
        ],
        out_specs=pl.BlockSpec((BLK, D), lambda g, bs, es, vs: (bs[g], 0)),
    )
    return pl.pallas_call(
        _gmm_body,
        grid_spec=grid_spec,
        out_shape=jax.ShapeDtypeStruct((A_PAD, D), _F32),
        compiler_params=pltpu.CompilerParams(
            vmem_limit_bytes=100 * 1024 * 1024),
    )(b_arr, e_arr, val_arr, xg, w1, w3, w2)


def _combine_body(h2_ref, ya_ref, yb_ref, tw_ref, out_ref):
    tw = tw_ref[...]
    out_ref[...] = (h2_ref[...] + tw[:, 0:1] * ya_ref[...]
                    + tw[:, 1:2] * yb_ref[...])


def _combine_call(h2, yg, tw):
    return pl.pallas_call(
        _combine_body,
        grid=(NSB,),
        in_specs=[
            pl.BlockSpec((BSQ, D), lambda i: (i, 0)),
            pl.BlockSpec((BSQ, D), lambda i: (i, 0)),
            pl.BlockSpec((BSQ, D), lambda i: (S // BSQ + i, 0)),
            pl.BlockSpec((BSQ, TOPK), lambda i: (i, 0)),
        ],
        out_specs=pl.BlockSpec((BSQ, D), lambda i: (i, 0)),
        out_shape=jax.ShapeDtypeStruct((S, D), _F32),
    )(h2, yg, yg, tw)


def _final_body(h_ref, w_ref, out_ref):
    out_ref[...] = _rms(h_ref[...], w_ref[...])


def _final_call(h, w):
    return pl.pallas_call(
        _final_body,
        out_shape=jax.ShapeDtypeStruct((S, D), _F32),
    )(h, w)


def _col_perm_q():
    import numpy as np
    n = np.arange(2 * QW)
    half, rest = n // QW, n % QW
    return (rest // HH) * HD + half * HH + rest % HH


def _col_perm_k():
    import numpy as np
    n = np.arange(2 * KW)
    half, rest = n // KW, n % KW
    return (rest // HH) * HD + half * HH + rest % HH


def _rope_tables():
    inv_freq = 1.0 / (THETA ** (jnp.arange(0, HD, 2).astype(_F32) / HD))
    freqs = jnp.arange(S, dtype=_F32)[:, None] * inv_freq[None, :]
    cosf, sinf = jnp.cos(freqs), jnp.sin(freqs)
    cq = jnp.tile(cosf, (1, 2 * QW // HH))
    sq = jnp.tile(sinf, (1, 2 * QW // HH))
    ck = jnp.tile(cosf, (1, 2 * KW // HH))
    sk = jnp.tile(sinf, (1, 2 * KW // HH))
    return cq, sq, ck, sk


@jax.jit
def _forward(x, params):
    cq, sq, ck, sk = _rope_tables()
    pq, pk = _col_perm_q(), _col_perm_k()
    h = x.reshape(S, D)
    for l in range(L):
        p = params['layer_%d' % l]
        q, k, v = _qkv_call(h, p['ln1'], p['wq'][:, pq], p['wk'][:, pk],
                            p['wv'], cq, sq, ck, sk)
        o = _attn_call(q, k, v)
        h2, r2, tw, eid, rank, cnt = _wo_router_call(
            h, o, p['wo'], p['ln2'], p['wg'])
        pos0, pos1, b_arr, e_arr, val_arr = _fixup_call(cnt, eid, rank)
        p0 = pos0.reshape(S)
        p1 = pos1.reshape(S)
        xg = _sc_dispatch(r2, p0, p1)
        y = _gmm_call(xg, p['w1'], p['w3'], p['w2'],
                      b_arr.reshape(G), e_arr.reshape(G),
                      val_arr.reshape(G))
        yg = _sc_combine(y, p0, p1)
        h = _combine_call(h2, yg, tw)
    return _final_call(h, params['final_ln']).reshape(1, S, D)


def kernel(input_ids, params):
    return _forward(input_ids, params)


# speedup vs baseline: 1.6422x; 1.0117x over previous
import functools

import jax
import jax.numpy as jnp
from jax.experimental import pallas as pl
from jax.experimental.pallas import tpu as pltpu

S, D = 2048, 1024
H, KV, HD = 16, 8, 64
E, TOPK, F = 8, 2, 1024
L = 2
EPS = 1e-6
THETA = 10000.0
HH = HD // 2

QW = H * HH
KW = KV * HH

BSQ = 512
NSB = S // BSQ

A = TOPK * S
BLK = 256
NB = A // BLK
G = NB + E - 1
A_PAD = G * BLK

SC_NC, SC_NS = 2, 16
SC_NW = SC_NC * SC_NS

_F32 = jnp.float32
_HI = jax.lax.Precision.HIGHEST


def _rms(x, w):
    return x * jax.lax.rsqrt(jnp.mean(x * x, axis=-1, keepdims=True) + EPS) * w


def _qkv_body(h_ref, ln1_ref, wq_ref, wk_ref, wv_ref, cq_ref, sq_ref,
              ck_ref, sk_ref, q_out, k_out, v_out):
    r = _rms(h_ref[...], ln1_ref[...])
    q = jnp.dot(r, wq_ref[...], preferred_element_type=_F32)
    k = jnp.dot(r, wk_ref[...], preferred_element_type=_F32)
    v = jnp.dot(r, wv_ref[...], preferred_element_type=_F32)
    qr = jnp.concatenate([-q[:, QW:], q[:, :QW]], axis=1)
    kr = jnp.concatenate([-k[:, KW:], k[:, :KW]], axis=1)
    q_out[...] = q * cq_ref[...] + qr * sq_ref[...]
    k_out[...] = k * ck_ref[...] + kr * sk_ref[...]
    v_out[...] = v


def _qkv_call(h, ln1, wq_p, wk_p, wv, cq, sq, ck, sk):
    return pl.pallas_call(
        _qkv_body,
        grid=(NSB,),
        in_specs=[
            pl.BlockSpec((BSQ, D), lambda i: (i, 0)),
            pl.BlockSpec((D,), lambda i: (0,)),
            pl.BlockSpec((D, 2 * QW), lambda i: (0, 0)),
            pl.BlockSpec((D, 2 * KW), lambda i: (0, 0)),
            pl.BlockSpec((D, KV * HD), lambda i: (0, 0)),
            pl.BlockSpec((BSQ, 2 * QW), lambda i: (i, 0)),
            pl.BlockSpec((BSQ, 2 * QW), lambda i: (i, 0)),
            pl.BlockSpec((BSQ, 2 * KW), lambda i: (i, 0)),
            pl.BlockSpec((BSQ, 2 * KW), lambda i: (i, 0)),
        ],
        out_specs=[
            pl.BlockSpec((BSQ, 2 * QW), lambda i: (i, 0)),
            pl.BlockSpec((BSQ, 2 * KW), lambda i: (i, 0)),
            pl.BlockSpec((BSQ, KV * HD), lambda i: (i, 0)),
        ],
        out_shape=[
            jax.ShapeDtypeStruct((S, 2 * QW), _F32),
            jax.ShapeDtypeStruct((S, 2 * KW), _F32),
            jax.ShapeDtypeStruct((S, KV * HD), _F32),
        ],
    )(h, ln1, wq_p, wk_p, wv, cq, sq, ck, sk)


BQ = 256
NQB = S // BQ


def _make_attn_body(qb0, kwid):
    def body(q_ref, k_ref, v_ref, o_ref):
        qb_i = pl.program_id(0)
        q = q_ref[...]
        k = k_ref[...]
        v = v_ref[...]
        rows = (jax.lax.broadcasted_iota(jnp.int32, (BQ, kwid), 0)
                + (qb0 + qb_i) * BQ)
        cols = jax.lax.broadcasted_iota(jnp.int32, (BQ, kwid), 1)
        bias = jnp.where(cols <= rows, 0.0, -1e9)
        for h in range(H):
            j = h // 2
            qh = jnp.concatenate(
                [q[:, h * HH:(h + 1) * HH],
                 q[:, QW + h * HH:QW + (h + 1) * HH]], axis=1)
            kh = jnp.concatenate(
                [k[:, j * HH:(j + 1) * HH],
                 k[:, KW + j * HH:KW + (j + 1) * HH]], axis=1)
            s = jnp.dot(qh, kh.T, preferred_element_type=_F32)
            s = s * (1.0 / (HD ** 0.5)) + bias
            p = jax.nn.softmax(s, axis=-1)
            o_ref[:, h * HD:(h + 1) * HD] = jnp.dot(
                p, v[:, j * HD:(j + 1) * HD], preferred_element_type=_F32)
    return body


QG = 2


def _attn_call(q, k, v):
    outs = []
    for g in range(NQB // QG):
        qb0 = g * QG
        kwid = (qb0 + QG) * BQ
        o_g = pl.pallas_call(
            _make_attn_body(qb0, kwid),
            grid=(QG,),
            in_specs=[
                pl.BlockSpec((BQ, 2 * QW), lambda qb, qb0=qb0: (qb0 + qb, 0)),
                pl.BlockSpec((kwid, 2 * KW), lambda qb: (0, 0)),
                pl.BlockSpec((kwid, KV * HD), lambda qb: (0, 0)),
            ],
            out_specs=pl.BlockSpec((BQ, H * HD), lambda qb: (qb, 0)),
            out_shape=jax.ShapeDtypeStruct((QG * BQ, H * HD), _F32),
            compiler_params=pltpu.CompilerParams(
                vmem_limit_bytes=100 * 1024 * 1024),
        )(q, k, v)
        outs.append(o_g)
    return jnp.concatenate(outs, axis=0)


def _wo_router_body(h_ref, o_ref, wo_ref, ln2_ref, wg_ref,
                    h2_out, r2_out, tw_out, eid_out, rank_out, cnt_out,
                    run_ref):
    i = pl.program_id(0)

    @pl.when(i == 0)
    def _():
        run_ref[...] = jnp.zeros((1, E), _F32)

    h2 = h_ref[...] + jnp.dot(o_ref[...], wo_ref[...],
                              preferred_element_type=_F32)
    h2_out[...] = h2
    r2 = _rms(h2, ln2_ref[...])
    r2_out[...] = r2
    logits = jnp.dot(r2, wg_ref[...], preferred_element_type=_F32)
    probs = jax.nn.softmax(logits, axis=-1)
    idx = jax.lax.broadcasted_iota(jnp.int32, (BSQ, E), 1)
    m1 = jnp.max(probs, axis=-1, keepdims=True)
    i1 = jnp.min(jnp.where(probs == m1, idx, E), axis=-1, keepdims=True)
    oh1 = (idx == i1).astype(_F32)
    rest = jnp.where(idx == i1, -jnp.inf, probs)
    m2 = jnp.max(rest, axis=-1, keepdims=True)
    i2 = jnp.min(jnp.where(rest == m2, idx, E), axis=-1, keepdims=True)
    oh2 = (idx == i2).astype(_F32)
    denom = m1 + m2
    tw_out[...] = jnp.concatenate([m1 / denom, m2 / denom], axis=1)
    eid_out[...] = jnp.concatenate([i1, i2], axis=1)

    oh12 = oh1 + oh2
    tr = jax.lax.broadcasted_iota(jnp.int32, (BSQ, BSQ), 0)
    tc = jax.lax.broadcasted_iota(jnp.int32, (BSQ, BSQ), 1)
    strict = (tc < tr).astype(_F32)
    pref = jax.lax.dot(strict, oh12, precision=_HI) + run_ref[...]
    rank1 = jnp.sum(oh1 * pref, axis=-1, keepdims=True)
    rank2 = jnp.sum(oh2 * pref, axis=-1, keepdims=True)
    rank_out[...] = jnp.concatenate([rank1, rank2], axis=1).astype(jnp.int32)
    run_ref[...] += jnp.sum(oh12, axis=0, keepdims=True)

    @pl.when(i == NSB - 1)
    def _():
        cnt_out[...] = run_ref[...].astype(jnp.int32)


def _wo_router_call(h, o, wo, ln2, wg):
    return pl.pallas_call(
        _wo_router_body,
        grid=(NSB,),
        in_specs=[
            pl.BlockSpec((BSQ, D), lambda i: (i, 0)),
            pl.BlockSpec((BSQ, H * HD), lambda i: (i, 0)),
            pl.BlockSpec((H * HD, D), lambda i: (0, 0)),
            pl.BlockSpec((D,), lambda i: (0,)),
            pl.BlockSpec((D, E), lambda i: (0, 0)),
        ],
        out_specs=[
            pl.BlockSpec((BSQ, D), lambda i: (i, 0)),
            pl.BlockSpec((BSQ, D), lambda i: (i, 0)),
            pl.BlockSpec((BSQ, TOPK), lambda i: (i, 0)),
            pl.BlockSpec((BSQ, TOPK), lambda i: (i, 0)),
            pl.BlockSpec((BSQ, TOPK), lambda i: (i, 0)),
            pl.BlockSpec((1, E), lambda i: (0, 0)),
        ],
        out_shape=[
            jax.ShapeDtypeStruct((S, D), _F32),
            jax.ShapeDtypeStruct((S, D), _F32),
            jax.ShapeDtypeStruct((S, TOPK), _F32),
            jax.ShapeDtypeStruct((S, TOPK), jnp.int32),
            jax.ShapeDtypeStruct((S, TOPK), jnp.int32),
            jax.ShapeDtypeStruct((1, E), jnp.int32),
        ],
        scratch_shapes=[pltpu.VMEM((1, E), _F32)],
    )(h, o, wo, ln2, wg)


def _fixup_body(cnt_ref, eid_ref, rank_ref,
                pos0_out, pos1_out, b_out, e_out, val_out):
    cnt = cnt_ref[...]
    pc = ((cnt + BLK - 1) // BLK) * BLK
    er = jax.lax.broadcasted_iota(jnp.int32, (E, E + 1), 0)
    jc = jax.lax.broadcasted_iota(jnp.int32, (E, E + 1), 1)
    mcum = (er < jc).astype(_F32)
    offp9 = jax.lax.dot(pc.astype(_F32), mcum,
                        precision=_HI).astype(jnp.int32)
    blkb = offp9 // BLK
    nbp = blkb[:, E:]
    g_col = jax.lax.broadcasted_iota(jnp.int32, (G, 1), 0)
    b_of_g = jnp.minimum(g_col, jnp.broadcast_to(nbp, (G, 1)) - 1)
    cmp = (jnp.broadcast_to(blkb, (G, E + 1)) <= b_of_g)
    e_of_g = jnp.clip(jnp.sum(cmp.astype(jnp.int32), axis=-1, keepdims=True)
                      - 1, 0, E - 1)
    b_out[...] = b_of_g
    e_out[...] = e_of_g
    val_out[...] = (g_col < jnp.broadcast_to(nbp, (G, 1))).astype(jnp.int32)

    te = jax.lax.broadcasted_iota(jnp.int32, (S, E), 1)
    off8b = jnp.broadcast_to(offp9[:, :E], (S, E))
    for k, out in ((0, pos0_out), (1, pos1_out)):
        ohk = te == eid_ref[:, k:k + 1]
        offsel = jnp.sum(jnp.where(ohk, off8b, 0), axis=-1, keepdims=True)
        out[...] = offsel + rank_ref[:, k:k + 1]


def _fixup_call(cnt, eid, rank):
    return pl.pallas_call(
        _fixup_body,
        out_shape=[
            jax.ShapeDtypeStruct((S, 1), jnp.int32),
            jax.ShapeDtypeStruct((S, 1), jnp.int32),
            jax.ShapeDtypeStruct((G, 1), jnp.int32),
            jax.ShapeDtypeStruct((G, 1), jnp.int32),
            jax.ShapeDtypeStruct((G, 1), jnp.int32),
        ],
    )(cnt, eid, rank)


def _make_sc_dispatch():
    from jax.experimental.pallas import tpu_sc as plsc
    per_w = S // SC_NW
    CH = 32
    nch = per_w // CH
    mesh = plsc.VectorSubcoreMesh(core_axis_name="c", subcore_axis_name="s",
                                  num_cores=SC_NC)

    @functools.partial(
        pl.kernel, mesh=mesh,
        out_type=jax.ShapeDtypeStruct((A_PAD, D), _F32),
        scratch_types=[
            pltpu.VMEM((CH,), jnp.int32),
            pltpu.VMEM((CH, D), _F32),
            pltpu.SemaphoreType.DMA,
        ],
    )
    def dispatch_k(r2_hbm, p0_hbm, p1_hbm, out_hbm, idx_v, rows_v, sem):
        wid = jax.lax.axis_index("s") * SC_NC + jax.lax.axis_index("c")
        for k in range(TOPK):
            idx_hbm = (p0_hbm, p1_hbm)[k]
            for c in range(nch):
                base = wid * per_w + c * CH
                pltpu.sync_copy(r2_hbm.at[pl.ds(base, CH)], rows_v)
                pltpu.sync_copy(idx_hbm.at[pl.ds(base, CH)], idx_v)
                pltpu.async_copy(rows_v, out_hbm.at[idx_v], sem).wait()

    return dispatch_k


def _make_sc_combine():
    from jax.experimental.pallas import tpu_sc as plsc
    per_w = S // SC_NW
    CH = 32
    nch = per_w // CH
    mesh = plsc.VectorSubcoreMesh(core_axis_name="c", subcore_axis_name="s",
                                  num_cores=SC_NC)

    @functools.partial(
        pl.kernel, mesh=mesh,
        out_type=jax.ShapeDtypeStruct((A, D), _F32),
        scratch_types=[
            pltpu.VMEM((CH,), jnp.int32),
            pltpu.VMEM((CH, D), _F32),
            pltpu.SemaphoreType.DMA,
        ],
    )
    def combine_k(y_hbm, p0_hbm, p1_hbm, out_hbm, idx_v, rows_v, sem):
        wid = jax.lax.axis_index("s") * SC_NC + jax.lax.axis_index("c")
        for k in range(TOPK):
            idx_hbm = (p0_hbm, p1_hbm)[k]
            for c in range(nch):
                base = wid * per_w + c * CH
                pltpu.sync_copy(idx_hbm.at[pl.ds(base, CH)], idx_v)
                pltpu.async_copy(y_hbm.at[idx_v], rows_v, sem).wait()
                pltpu.sync_copy(rows_v, out_hbm.at[pl.ds(k * S + base, CH)])

    return combine_k


_sc_cache = {}


def _sc_dispatch(r2, p0, p1):
    if 'd' not in _sc_cache:
        _sc_cache['d'] = _make_sc_dispatch()
    return _sc_cache['d'](r2, p0, p1)


def _sc_combine(y, p0, p1):
    if 'c' not in _sc_cache:
        _sc_cache['c'] = _make_sc_combine()
    return _sc_cache['c'](y, p0, p1)


def _gmm_body(b_arr, e_arr, val_arr,
              x_ref, w1_ref, w3_ref, w2_ref, out_ref):
    g = pl.program_id(0)

    @pl.when(val_arr[g] > 0)
    def _():
        x = x_ref[...]
        a = jnp.dot(x, w1_ref[0], preferred_element_type=_F32)
        bb = jnp.dot(x, w3_ref[0], preferred_element_type=_F32)
        out_ref[...] = jnp.dot(a * jax.nn.sigmoid(a) * bb, w2_ref[0],
                               preferred_element_type=_F32)


def _gmm_call(xg, w1, w3, w2, b_arr, e_arr, val_arr):
    grid_spec = pltpu.PrefetchScalarGridSpec(
        num_scalar_prefetch=3,
        grid=(G,),
        in_specs=[
            pl.BlockSpec((BLK, D), lambda g, bs, es, vs: (bs[g], 0)),
            pl.BlockSpec((1, D, F), lambda g, bs, es, vs: (es[g], 0, 0)),
            pl.BlockSpec((1, D, F), lambda g, bs, es, vs: (es[g], 0, 0)),
            pl.BlockSpec((1, F, D), lambda g, bs, es, vs: (es[g], 0, 0)),
        ],
        out_specs=pl.BlockSpec((BLK, D), lambda g, bs, es, vs: (bs[g], 0)),
    )
    return pl.pallas_call(
        _gmm_body,
        grid_spec=grid_spec,
        out_shape=jax.ShapeDtypeStruct((A_PAD, D), _F32),
        compiler_params=pltpu.CompilerParams(
            vmem_limit_bytes=100 * 1024 * 1024),
    )(b_arr, e_arr, val_arr, xg, w1, w3, w2)


def _combine_body_plain(h2_ref, ya_ref, yb_ref, tw_ref, out_ref):
    tw = tw_ref[...]
    out_ref[...] = (h2_ref[...] + tw[:, 0:1] * ya_ref[...]
                    + tw[:, 1:2] * yb_ref[...])


def _combine_body_final(h2_ref, ya_ref, yb_ref, tw_ref, fw_ref, out_ref):
    tw = tw_ref[...]
    h3 = (h2_ref[...] + tw[:, 0:1] * ya_ref[...]
          + tw[:, 1:2] * yb_ref[...])
    out_ref[...] = _rms(h3, fw_ref[...])


def _combine_call(h2, yg, tw, final_w=None):
    specs = [
        pl.BlockSpec((BSQ, D), lambda i: (i, 0)),
        pl.BlockSpec((BSQ, D), lambda i: (i, 0)),
        pl.BlockSpec((BSQ, D), lambda i: (S // BSQ + i, 0)),
        pl.BlockSpec((BSQ, TOPK), lambda i: (i, 0)),
    ]
    args = (h2, yg, yg, tw)
    body = _combine_body_plain
    if final_w is not None:
        specs.append(pl.BlockSpec((D,), lambda i: (0,)))
        args = args + (final_w,)
        body = _combine_body_final
    return pl.pallas_call(
        body,
        grid=(NSB,),
        in_specs=specs,
        out_specs=pl.BlockSpec((BSQ, D), lambda i: (i, 0)),
        out_shape=jax.ShapeDtypeStruct((S, D), _F32),
    )(*args)


def _final_body(h_ref, w_ref, out_ref):
    out_ref[...] = _rms(h_ref[...], w_ref[...])


def _final_call(h, w):
    return pl.pallas_call(
        _final_body,
        out_shape=jax.ShapeDtypeStruct((S, D), _F32),
    )(h, w)


def _col_perm_q():
    import numpy as np
    n = np.arange(2 * QW)
    half, rest = n // QW, n % QW
    return (rest // HH) * HD + half * HH + rest % HH


def _col_perm_k():
    import numpy as np
    n = np.arange(2 * KW)
    half, rest = n // KW, n % KW
    return (rest // HH) * HD + half * HH + rest % HH


def _rope_tables():
    inv_freq = 1.0 / (THETA ** (jnp.arange(0, HD, 2).astype(_F32) / HD))
    freqs = jnp.arange(S, dtype=_F32)[:, None] * inv_freq[None, :]
    cosf, sinf = jnp.cos(freqs), jnp.sin(freqs)
    cq = jnp.tile(cosf, (1, 2 * QW // HH))
    sq = jnp.tile(sinf, (1, 2 * QW // HH))
    ck = jnp.tile(cosf, (1, 2 * KW // HH))
    sk = jnp.tile(sinf, (1, 2 * KW // HH))
    return cq, sq, ck, sk


@jax.jit
def _forward(x, params):
    cq, sq, ck, sk = _rope_tables()
    pq, pk = _col_perm_q(), _col_perm_k()
    h = x.reshape(S, D)
    for l in range(L):
        p = params['layer_%d' % l]
        q, k, v = _qkv_call(h, p['ln1'], p['wq'][:, pq], p['wk'][:, pk],
                            p['wv'], cq, sq, ck, sk)
        o = _attn_call(q, k, v)
        h2, r2, tw, eid, rank, cnt = _wo_router_call(
            h, o, p['wo'], p['ln2'], p['wg'])
        pos0, pos1, b_arr, e_arr, val_arr = _fixup_call(cnt, eid, rank)
        p0 = pos0.reshape(S)
        p1 = pos1.reshape(S)
        xg = _sc_dispatch(r2, p0, p1)
        y = _gmm_call(xg, p['w1'], p['w3'], p['w2'],
                      b_arr.reshape(G), e_arr.reshape(G),
                      val_arr.reshape(G))
        yg = _sc_combine(y, p0, p1)
        if l == L - 1:
            h = _combine_call(h2, yg, tw, params['final_ln'])
        else:
            h = _combine_call(h2, yg, tw)
    return h.reshape(1, S, D)


def kernel(input_ids, params):
    return _forward(input_ids, params)


# speedup vs baseline: 1.6776x; 1.0216x over previous
import functools

import jax
import jax.numpy as jnp
from jax.experimental import pallas as pl
from jax.experimental.pallas import tpu as pltpu

S, D = 2048, 1024
H, KV, HD = 16, 8, 64
E, TOPK, F = 8, 2, 1024
L = 2
EPS = 1e-6
THETA = 10000.0
HH = HD // 2

QW = H * HH
KW = KV * HH

BSQ = 512
NSB = S // BSQ

A = TOPK * S
BLK = 512
NB = A // BLK
G = NB + E - 1
A_PAD = G * BLK

SC_NC, SC_NS = 2, 16
SC_NW = SC_NC * SC_NS

_F32 = jnp.float32
_HI = jax.lax.Precision.HIGHEST


def _rms(x, w):
    return x * jax.lax.rsqrt(jnp.mean(x * x, axis=-1, keepdims=True) + EPS) * w


def _qkv_body(h_ref, ln1_ref, wq_ref, wk_ref, wv_ref, cq_ref, sq_ref,
              ck_ref, sk_ref, q_out, k_out, v_out):
    r = _rms(h_ref[...], ln1_ref[...])
    q = jnp.dot(r, wq_ref[...], preferred_element_type=_F32)
    k = jnp.dot(r, wk_ref[...], preferred_element_type=_F32)
    v = jnp.dot(r, wv_ref[...], preferred_element_type=_F32)
    qr = jnp.concatenate([-q[:, QW:], q[:, :QW]], axis=1)
    kr = jnp.concatenate([-k[:, KW:], k[:, :KW]], axis=1)
    q_out[...] = q * cq_ref[...] + qr * sq_ref[...]
    k_out[...] = k * ck_ref[...] + kr * sk_ref[...]
    v_out[...] = v


def _qkv_call(h, ln1, wq_p, wk_p, wv, cq, sq, ck, sk):
    return pl.pallas_call(
        _qkv_body,
        grid=(NSB,),
        in_specs=[
            pl.BlockSpec((BSQ, D), lambda i: (i, 0)),
            pl.BlockSpec((D,), lambda i: (0,)),
            pl.BlockSpec((D, 2 * QW), lambda i: (0, 0)),
            pl.BlockSpec((D, 2 * KW), lambda i: (0, 0)),
            pl.BlockSpec((D, KV * HD), lambda i: (0, 0)),
            pl.BlockSpec((BSQ, 2 * QW), lambda i: (i, 0)),
            pl.BlockSpec((BSQ, 2 * QW), lambda i: (i, 0)),
            pl.BlockSpec((BSQ, 2 * KW), lambda i: (i, 0)),
            pl.BlockSpec((BSQ, 2 * KW), lambda i: (i, 0)),
        ],
        out_specs=[
            pl.BlockSpec((BSQ, 2 * QW), lambda i: (i, 0)),
            pl.BlockSpec((BSQ, 2 * KW), lambda i: (i, 0)),
            pl.BlockSpec((BSQ, KV * HD), lambda i: (i, 0)),
        ],
        out_shape=[
            jax.ShapeDtypeStruct((S, 2 * QW), _F32),
            jax.ShapeDtypeStruct((S, 2 * KW), _F32),
            jax.ShapeDtypeStruct((S, KV * HD), _F32),
        ],
    )(h, ln1, wq_p, wk_p, wv, cq, sq, ck, sk)


BQ = 256
NQB = S // BQ


def _make_attn_body(qb0, kwid):
    def body(q_ref, k_ref, v_ref, o_ref):
        qb_i = pl.program_id(0)
        q = q_ref[...]
        k = k_ref[...]
        v = v_ref[...]
        rows = (jax.lax.broadcasted_iota(jnp.int32, (BQ, kwid), 0)
                + (qb0 + qb_i) * BQ)
        cols = jax.lax.broadcasted_iota(jnp.int32, (BQ, kwid), 1)
        bias = jnp.where(cols <= rows, 0.0, -1e9)
        for h in range(H):
            j = h // 2
            qh = jnp.concatenate(
                [q[:, h * HH:(h + 1) * HH],
                 q[:, QW + h * HH:QW + (h + 1) * HH]], axis=1)
            kh = jnp.concatenate(
                [k[:, j * HH:(j + 1) * HH],
                 k[:, KW + j * HH:KW + (j + 1) * HH]], axis=1)
            s = jnp.dot(qh, kh.T, preferred_element_type=_F32)
            s = s * (1.0 / (HD ** 0.5)) + bias
            p = jax.nn.softmax(s, axis=-1)
            o_ref[:, h * HD:(h + 1) * HD] = jnp.dot(
                p, v[:, j * HD:(j + 1) * HD], preferred_element_type=_F32)
    return body


QG = 2


def _attn_call(q, k, v):
    outs = []
    for g in range(NQB // QG):
        qb0 = g * QG
        kwid = (qb0 + QG) * BQ
        o_g = pl.pallas_call(
            _make_attn_body(qb0, kwid),
            grid=(QG,),
            in_specs=[
                pl.BlockSpec((BQ, 2 * QW), lambda qb, qb0=qb0: (qb0 + qb, 0)),
                pl.BlockSpec((kwid, 2 * KW), lambda qb: (0, 0)),
                pl.BlockSpec((kwid, KV * HD), lambda qb: (0, 0)),
            ],
            out_specs=pl.BlockSpec((BQ, H * HD), lambda qb: (qb, 0)),
            out_shape=jax.ShapeDtypeStruct((QG * BQ, H * HD), _F32),
            compiler_params=pltpu.CompilerParams(
                vmem_limit_bytes=100 * 1024 * 1024),
        )(q, k, v)
        outs.append(o_g)
    return jnp.concatenate(outs, axis=0)


def _wo_router_body(h_ref, o_ref, wo_ref, ln2_ref, wg_ref,
                    h2_out, r2_out, tw_out, eid_out, rank_out, cnt_out,
                    run_ref):
    i = pl.program_id(0)

    @pl.when(i == 0)
    def _():
        run_ref[...] = jnp.zeros((1, E), _F32)

    h2 = h_ref[...] + jnp.dot(o_ref[...], wo_ref[...],
                              preferred_element_type=_F32)
    h2_out[...] = h2
    r2 = _rms(h2, ln2_ref[...])
    r2_out[...] = r2
    logits = jnp.dot(r2, wg_ref[...], preferred_element_type=_F32)
    probs = jax.nn.softmax(logits, axis=-1)
    idx = jax.lax.broadcasted_iota(jnp.int32, (BSQ, E), 1)
    m1 = jnp.max(probs, axis=-1, keepdims=True)
    i1 = jnp.min(jnp.where(probs == m1, idx, E), axis=-1, keepdims=True)
    oh1 = (idx == i1).astype(_F32)
    rest = jnp.where(idx == i1, -jnp.inf, probs)
    m2 = jnp.max(rest, axis=-1, keepdims=True)
    i2 = jnp.min(jnp.where(rest == m2, idx, E), axis=-1, keepdims=True)
    oh2 = (idx == i2).astype(_F32)
    denom = m1 + m2
    tw_out[...] = jnp.concatenate([m1 / denom, m2 / denom], axis=1)
    eid_out[...] = jnp.concatenate([i1, i2], axis=1)

    oh12 = oh1 + oh2
    tr = jax.lax.broadcasted_iota(jnp.int32, (BSQ, BSQ), 0)
    tc = jax.lax.broadcasted_iota(jnp.int32, (BSQ, BSQ), 1)
    strict = (tc < tr).astype(_F32)
    pref = jax.lax.dot(strict, oh12, precision=_HI) + run_ref[...]
    rank1 = jnp.sum(oh1 * pref, axis=-1, keepdims=True)
    rank2 = jnp.sum(oh2 * pref, axis=-1, keepdims=True)
    rank_out[...] = jnp.concatenate([rank1, rank2], axis=1).astype(jnp.int32)
    run_ref[...] += jnp.sum(oh12, axis=0, keepdims=True)

    @pl.when(i == NSB - 1)
    def _():
        cnt_out[...] = run_ref[...].astype(jnp.int32)


def _wo_router_call(h, o, wo, ln2, wg):
    return pl.pallas_call(
        _wo_router_body,
        grid=(NSB,),
        in_specs=[
            pl.BlockSpec((BSQ, D), lambda i: (i, 0)),
            pl.BlockSpec((BSQ, H * HD), lambda i: (i, 0)),
            pl.BlockSpec((H * HD, D), lambda i: (0, 0)),
            pl.BlockSpec((D,), lambda i: (0,)),
            pl.BlockSpec((D, E), lambda i: (0, 0)),
        ],
        out_specs=[
            pl.BlockSpec((BSQ, D), lambda i: (i, 0)),
            pl.BlockSpec((BSQ, D), lambda i: (i, 0)),
            pl.BlockSpec((BSQ, TOPK), lambda i: (i, 0)),
            pl.BlockSpec((BSQ, TOPK), lambda i: (i, 0)),
            pl.BlockSpec((BSQ, TOPK), lambda i: (i, 0)),
            pl.BlockSpec((1, E), lambda i: (0, 0)),
        ],
        out_shape=[
            jax.ShapeDtypeStruct((S, D), _F32),
            jax.ShapeDtypeStruct((S, D), _F32),
            jax.ShapeDtypeStruct((S, TOPK), _F32),
            jax.ShapeDtypeStruct((S, TOPK), jnp.int32),
            jax.ShapeDtypeStruct((S, TOPK), jnp.int32),
            jax.ShapeDtypeStruct((1, E), jnp.int32),
        ],
        scratch_shapes=[pltpu.VMEM((1, E), _F32)],
    )(h, o, wo, ln2, wg)


def _fixup_body(cnt_ref, eid_ref, rank_ref,
                pos0_out, pos1_out, b_out, e_out, val_out):
    cnt = cnt_ref[...]
    pc = ((cnt + BLK - 1) // BLK) * BLK
    er = jax.lax.broadcasted_iota(jnp.int32, (E, E + 1), 0)
    jc = jax.lax.broadcasted_iota(jnp.int32, (E, E + 1), 1)
    mcum = (er < jc).astype(_F32)
    offp9 = jax.lax.dot(pc.astype(_F32), mcum,
                        precision=_HI).astype(jnp.int32)
    blkb = offp9 // BLK
    nbp = blkb[:, E:]
    g_col = jax.lax.broadcasted_iota(jnp.int32, (G, 1), 0)
    b_of_g = jnp.minimum(g_col, jnp.broadcast_to(nbp, (G, 1)) - 1)
    cmp = (jnp.broadcast_to(blkb, (G, E + 1)) <= b_of_g)
    e_of_g = jnp.clip(jnp.sum(cmp.astype(jnp.int32), axis=-1, keepdims=True)
                      - 1, 0, E - 1)
    b_out[...] = b_of_g
    e_out[...] = e_of_g
    val_out[...] = (g_col < jnp.broadcast_to(nbp, (G, 1))).astype(jnp.int32)

    te = jax.lax.broadcasted_iota(jnp.int32, (S, E), 1)
    off8b = jnp.broadcast_to(offp9[:, :E], (S, E))
    for k, out in ((0, pos0_out), (1, pos1_out)):
        ohk = te == eid_ref[:, k:k + 1]
        offsel = jnp.sum(jnp.where(ohk, off8b, 0), axis=-1, keepdims=True)
        out[...] = offsel + rank_ref[:, k:k + 1]


def _fixup_call(cnt, eid, rank):
    return pl.pallas_call(
        _fixup_body,
        out_shape=[
            jax.ShapeDtypeStruct((S, 1), jnp.int32),
            jax.ShapeDtypeStruct((S, 1), jnp.int32),
            jax.ShapeDtypeStruct((G, 1), jnp.int32),
            jax.ShapeDtypeStruct((G, 1), jnp.int32),
            jax.ShapeDtypeStruct((G, 1), jnp.int32),
        ],
    )(cnt, eid, rank)


def _make_sc_dispatch():
    from jax.experimental.pallas import tpu_sc as plsc
    per_w = S // SC_NW
    CH = 32
    nch = per_w // CH
    mesh = plsc.VectorSubcoreMesh(core_axis_name="c", subcore_axis_name="s",
                                  num_cores=SC_NC)

    @functools.partial(
        pl.kernel, mesh=mesh,
        out_type=jax.ShapeDtypeStruct((A_PAD, D), _F32),
        scratch_types=[
            pltpu.VMEM((CH,), jnp.int32),
            pltpu.VMEM((CH, D), _F32),
            pltpu.SemaphoreType.DMA,
        ],
    )
    def dispatch_k(r2_hbm, p0_hbm, p1_hbm, out_hbm, idx_v, rows_v, sem):
        wid = jax.lax.axis_index("s") * SC_NC + jax.lax.axis_index("c")
        for k in range(TOPK):
            idx_hbm = (p0_hbm, p1_hbm)[k]
            for c in range(nch):
                base = wid * per_w + c * CH
                pltpu.sync_copy(r2_hbm.at[pl.ds(base, CH)], rows_v)
                pltpu.sync_copy(idx_hbm.at[pl.ds(base, CH)], idx_v)
                pltpu.async_copy(rows_v, out_hbm.at[idx_v], sem).wait()

    return dispatch_k


def _make_sc_combine():
    from jax.experimental.pallas import tpu_sc as plsc
    per_w = S // SC_NW
    CH = 32
    nch = per_w // CH
    mesh = plsc.VectorSubcoreMesh(core_axis_name="c", subcore_axis_name="s",
                                  num_cores=SC_NC)

    @functools.partial(
        pl.kernel, mesh=mesh,
        out_type=jax.ShapeDtypeStruct((A, D), _F32),
        scratch_types=[
            pltpu.VMEM((CH,), jnp.int32),
            pltpu.VMEM((CH, D), _F32),
            pltpu.SemaphoreType.DMA,
        ],
    )
    def combine_k(y_hbm, p0_hbm, p1_hbm, out_hbm, idx_v, rows_v, sem):
        wid = jax.lax.axis_index("s") * SC_NC + jax.lax.axis_index("c")
        for k in range(TOPK):
            idx_hbm = (p0_hbm, p1_hbm)[k]
            for c in range(nch):
                base = wid * per_w + c * CH
                pltpu.sync_copy(idx_hbm.at[pl.ds(base, CH)], idx_v)
                pltpu.async_copy(y_hbm.at[idx_v], rows_v, sem).wait()
                pltpu.sync_copy(rows_v, out_hbm.at[pl.ds(k * S + base, CH)])

    return combine_k


_sc_cache = {}


def _sc_dispatch(r2, p0, p1):
    if 'd' not in _sc_cache:
        _sc_cache['d'] = _make_sc_dispatch()
    return _sc_cache['d'](r2, p0, p1)


def _sc_combine(y, p0, p1):
    if 'c' not in _sc_cache:
        _sc_cache['c'] = _make_sc_combine()
    return _sc_cache['c'](y, p0, p1)


def _gmm_body(b_arr, e_arr, val_arr,
              x_ref, w1_ref, w3_ref, w2_ref, out_ref):
    g = pl.program_id(0)

    @pl.when(val_arr[g] > 0)
    def _():
        x = x_ref[...]
        a = jnp.dot(x, w1_ref[0], preferred_element_type=_F32)
        bb = jnp.dot(x, w3_ref[0], preferred_element_type=_F32)
        out_ref[...] = jnp.dot(a * jax.nn.sigmoid(a) * bb, w2_ref[0],
                               preferred_element_type=_F32)


def _gmm_call(xg, w1, w3, w2, b_arr, e_arr, val_arr):
    grid_spec = pltpu.PrefetchScalarGridSpec(
        num_scalar_prefetch=3,
        grid=(G,),
        in_specs=[
            pl.BlockSpec((BLK, D), lambda g, bs, es, vs: (bs[g], 0)),
            pl.BlockSpec((1, D, F), lambda g, bs, es, vs: (es[g], 0, 0)),
            pl.BlockSpec((1, D, F), lambda g, bs, es, vs: (es[g], 0, 0)),
            pl.BlockSpec((1, F, D), lambda g, bs, es, vs: (es[g], 0, 0)),
        ],
        out_specs=pl.BlockSpec((BLK, D), lambda g, bs, es, vs: (bs[g], 0)),
    )
    return pl.pallas_call(
        _gmm_body,
        grid_spec=grid_spec,
        out_shape=jax.ShapeDtypeStruct((A_PAD, D), _F32),
        compiler_params=pltpu.CompilerParams(
            vmem_limit_bytes=100 * 1024 * 1024),
    )(b_arr, e_arr, val_arr, xg, w1, w3, w2)


def _combine_body_plain(h2_ref, ya_ref, yb_ref, tw_ref, out_ref):
    tw = tw_ref[...]
    out_ref[...] = (h2_ref[...] + tw[:, 0:1] * ya_ref[...]
                    + tw[:, 1:2] * yb_ref[...])


def _combine_body_final(h2_ref, ya_ref, yb_ref, tw_ref, fw_ref, out_ref):
    tw = tw_ref[...]
    h3 = (h2_ref[...] + tw[:, 0:1] * ya_ref[...]
          + tw[:, 1:2] * yb_ref[...])
    out_ref[...] = _rms(h3, fw_ref[...])


def _combine_call(h2, yg, tw, final_w=None):
    specs = [
        pl.BlockSpec((BSQ, D), lambda i: (i, 0)),
        pl.BlockSpec((BSQ, D), lambda i: (i, 0)),
        pl.BlockSpec((BSQ, D), lambda i: (S // BSQ + i, 0)),
        pl.BlockSpec((BSQ, TOPK), lambda i: (i, 0)),
    ]
    args = (h2, yg, yg, tw)
    body = _combine_body_plain
    if final_w is not None:
        specs.append(pl.BlockSpec((D,), lambda i: (0,)))
        args = args + (final_w,)
        body = _combine_body_final
    return pl.pallas_call(
        body,
        grid=(NSB,),
        in_specs=specs,
        out_specs=pl.BlockSpec((BSQ, D), lambda i: (i, 0)),
        out_shape=jax.ShapeDtypeStruct((S, D), _F32),
    )(*args)


def _final_body(h_ref, w_ref, out_ref):
    out_ref[...] = _rms(h_ref[...], w_ref[...])


def _final_call(h, w):
    return pl.pallas_call(
        _final_body,
        out_shape=jax.ShapeDtypeStruct((S, D), _F32),
    )(h, w)


def _col_perm_q():
    import numpy as np
    n = np.arange(2 * QW)
    half, rest = n // QW, n % QW
    return (rest // HH) * HD + half * HH + rest % HH


def _col_perm_k():
    import numpy as np
    n = np.arange(2 * KW)
    half, rest = n // KW, n % KW
    return (rest // HH) * HD + half * HH + rest % HH


def _rope_tables():
    inv_freq = 1.0 / (THETA ** (jnp.arange(0, HD, 2).astype(_F32) / HD))
    freqs = jnp.arange(S, dtype=_F32)[:, None] * inv_freq[None, :]
    cosf, sinf = jnp.cos(freqs), jnp.sin(freqs)
    cq = jnp.tile(cosf, (1, 2 * QW // HH))
    sq = jnp.tile(sinf, (1, 2 * QW // HH))
    ck = jnp.tile(cosf, (1, 2 * KW // HH))
    sk = jnp.tile(sinf, (1, 2 * KW // HH))
    return cq, sq, ck, sk


@jax.jit
def _forward(x, params):
    cq, sq, ck, sk = _rope_tables()
    pq, pk = _col_perm_q(), _col_perm_k()
    h = x.reshape(S, D)
    for l in range(L):
        p = params['layer_%d' % l]
        q, k, v = _qkv_call(h, p['ln1'], p['wq'][:, pq], p['wk'][:, pk],
                            p['wv'], cq, sq, ck, sk)
        o = _attn_call(q, k, v)
        h2, r2, tw, eid, rank, cnt = _wo_router_call(
            h, o, p['wo'], p['ln2'], p['wg'])
        pos0, pos1, b_arr, e_arr, val_arr = _fixup_call(cnt, eid, rank)
        p0 = pos0.reshape(S)
        p1 = pos1.reshape(S)
        xg = _sc_dispatch(r2, p0, p1)
        y = _gmm_call(xg, p['w1'], p['w3'], p['w2'],
                      b_arr.reshape(G), e_arr.reshape(G),
                      val_arr.reshape(G))
        yg = _sc_combine(y, p0, p1)
        if l == L - 1:
            h = _combine_call(h2, yg, tw, params['final_ln'])
        else:
            h = _combine_call(h2, yg, tw)
    return h.reshape(1, S, D)


def kernel(input_ids, params):
    return _forward(input_ids, params)


# speedup vs baseline: 1.7204x; 1.0255x over previous
import functools

import jax
import jax.numpy as jnp
from jax.experimental import pallas as pl
from jax.experimental.pallas import tpu as pltpu

S, D = 2048, 1024
H, KV, HD = 16, 8, 64
E, TOPK, F = 8, 2, 1024
L = 2
EPS = 1e-6
THETA = 10000.0
HH = HD // 2

QW = H * HH
KW = KV * HH

BSQ = 512
NSB = S // BSQ

A = TOPK * S
BLK = 768
G = (E - 1) + (A - (E - 1) + BLK - 1) // BLK
NB = G
A_PAD = G * BLK

SC_NC, SC_NS = 2, 16
SC_NW = SC_NC * SC_NS

_F32 = jnp.float32
_HI = jax.lax.Precision.HIGHEST


def _rms(x, w):
    return x * jax.lax.rsqrt(jnp.mean(x * x, axis=-1, keepdims=True) + EPS) * w


def _qkv_body(h_ref, ln1_ref, wq_ref, wk_ref, wv_ref, cq_ref, sq_ref,
              ck_ref, sk_ref, q_out, k_out, v_out):
    r = _rms(h_ref[...], ln1_ref[...])
    q = jnp.dot(r, wq_ref[...], preferred_element_type=_F32)
    k = jnp.dot(r, wk_ref[...], preferred_element_type=_F32)
    v = jnp.dot(r, wv_ref[...], preferred_element_type=_F32)
    qr = jnp.concatenate([-q[:, QW:], q[:, :QW]], axis=1)
    kr = jnp.concatenate([-k[:, KW:], k[:, :KW]], axis=1)
    q_out[...] = q * cq_ref[...] + qr * sq_ref[...]
    k_out[...] = k * ck_ref[...] + kr * sk_ref[...]
    v_out[...] = v


def _qkv_call(h, ln1, wq_p, wk_p, wv, cq, sq, ck, sk):
    return pl.pallas_call(
        _qkv_body,
        grid=(NSB,),
        in_specs=[
            pl.BlockSpec((BSQ, D), lambda i: (i, 0)),
            pl.BlockSpec((D,), lambda i: (0,)),
            pl.BlockSpec((D, 2 * QW), lambda i: (0, 0)),
            pl.BlockSpec((D, 2 * KW), lambda i: (0, 0)),
            pl.BlockSpec((D, KV * HD), lambda i: (0, 0)),
            pl.BlockSpec((BSQ, 2 * QW), lambda i: (i, 0)),
            pl.BlockSpec((BSQ, 2 * QW), lambda i: (i, 0)),
            pl.BlockSpec((BSQ, 2 * KW), lambda i: (i, 0)),
            pl.BlockSpec((BSQ, 2 * KW), lambda i: (i, 0)),
        ],
        out_specs=[
            pl.BlockSpec((BSQ, 2 * QW), lambda i: (i, 0)),
            pl.BlockSpec((BSQ, 2 * KW), lambda i: (i, 0)),
            pl.BlockSpec((BSQ, KV * HD), lambda i: (i, 0)),
        ],
        out_shape=[
            jax.ShapeDtypeStruct((S, 2 * QW), _F32),
            jax.ShapeDtypeStruct((S, 2 * KW), _F32),
            jax.ShapeDtypeStruct((S, KV * HD), _F32),
        ],
    )(h, ln1, wq_p, wk_p, wv, cq, sq, ck, sk)


BQ = 256
NQB = S // BQ


def _make_attn_body(qb0, kwid):
    def body(q_ref, k_ref, v_ref, o_ref):
        qb_i = pl.program_id(0)
        q = q_ref[...]
        k = k_ref[...]
        v = v_ref[...]
        rows = (jax.lax.broadcasted_iota(jnp.int32, (BQ, kwid), 0)
                + (qb0 + qb_i) * BQ)
        cols = jax.lax.broadcasted_iota(jnp.int32, (BQ, kwid), 1)
        bias = jnp.where(cols <= rows, 0.0, -1e9)
        for h in range(H):
            j = h // 2
            qh = jnp.concatenate(
                [q[:, h * HH:(h + 1) * HH],
                 q[:, QW + h * HH:QW + (h + 1) * HH]], axis=1)
            kh = jnp.concatenate(
                [k[:, j * HH:(j + 1) * HH],
                 k[:, KW + j * HH:KW + (j + 1) * HH]], axis=1)
            s = jnp.dot(qh, kh.T, preferred_element_type=_F32)
            s = s * (1.0 / (HD ** 0.5)) + bias
            p = jax.nn.softmax(s, axis=-1)
            o_ref[:, h * HD:(h + 1) * HD] = jnp.dot(
                p, v[:, j * HD:(j + 1) * HD], preferred_element_type=_F32)
    return body


QG = 2


def _attn_call(q, k, v):
    outs = []
    for g in range(NQB // QG):
        qb0 = g * QG
        kwid = (qb0 + QG) * BQ
        o_g = pl.pallas_call(
            _make_attn_body(qb0, kwid),
            grid=(QG,),
            in_specs=[
                pl.BlockSpec((BQ, 2 * QW), lambda qb, qb0=qb0: (qb0 + qb, 0)),
                pl.BlockSpec((kwid, 2 * KW), lambda qb: (0, 0)),
                pl.BlockSpec((kwid, KV * HD), lambda qb: (0, 0)),
            ],
            out_specs=pl.BlockSpec((BQ, H * HD), lambda qb: (qb, 0)),
            out_shape=jax.ShapeDtypeStruct((QG * BQ, H * HD), _F32),
            compiler_params=pltpu.CompilerParams(
                vmem_limit_bytes=100 * 1024 * 1024),
        )(q, k, v)
        outs.append(o_g)
    return jnp.concatenate(outs, axis=0)


def _wo_router_body(h_ref, o_ref, wo_ref, ln2_ref, wg_ref,
                    h2_out, r2_out, tw_out, eid_out, rank_out, cnt_out,
                    run_ref):
    i = pl.program_id(0)

    @pl.when(i == 0)
    def _():
        run_ref[...] = jnp.zeros((1, E), _F32)

    h2 = h_ref[...] + jnp.dot(o_ref[...], wo_ref[...],
                              preferred_element_type=_F32)
    h2_out[...] = h2
    r2 = _rms(h2, ln2_ref[...])
    r2_out[...] = r2
    logits = jnp.dot(r2, wg_ref[...], preferred_element_type=_F32)
    probs = jax.nn.softmax(logits, axis=-1)
    idx = jax.lax.broadcasted_iota(jnp.int32, (BSQ, E), 1)
    m1 = jnp.max(probs, axis=-1, keepdims=True)
    i1 = jnp.min(jnp.where(probs == m1, idx, E), axis=-1, keepdims=True)
    oh1 = (idx == i1).astype(_F32)
    rest = jnp.where(idx == i1, -jnp.inf, probs)
    m2 = jnp.max(rest, axis=-1, keepdims=True)
    i2 = jnp.min(jnp.where(rest == m2, idx, E), axis=-1, keepdims=True)
    oh2 = (idx == i2).astype(_F32)
    denom = m1 + m2
    tw_out[...] = jnp.concatenate([m1 / denom, m2 / denom], axis=1)
    eid_out[...] = jnp.concatenate([i1, i2], axis=1)

    oh12 = oh1 + oh2
    tr = jax.lax.broadcasted_iota(jnp.int32, (BSQ, BSQ), 0)
    tc = jax.lax.broadcasted_iota(jnp.int32, (BSQ, BSQ), 1)
    strict = (tc < tr).astype(_F32)
    pref = jax.lax.dot(strict, oh12, precision=_HI) + run_ref[...]
    rank1 = jnp.sum(oh1 * pref, axis=-1, keepdims=True)
    rank2 = jnp.sum(oh2 * pref, axis=-1, keepdims=True)
    rank_out[...] = jnp.concatenate([rank1, rank2], axis=1).astype(jnp.int32)
    run_ref[...] += jnp.sum(oh12, axis=0, keepdims=True)

    @pl.when(i == NSB - 1)
    def _():
        cnt_out[...] = run_ref[...].astype(jnp.int32)


def _wo_router_call(h, o, wo, ln2, wg):
    return pl.pallas_call(
        _wo_router_body,
        grid=(NSB,),
        in_specs=[
            pl.BlockSpec((BSQ, D), lambda i: (i, 0)),
            pl.BlockSpec((BSQ, H * HD), lambda i: (i, 0)),
            pl.BlockSpec((H * HD, D), lambda i: (0, 0)),
            pl.BlockSpec((D,), lambda i: (0,)),
            pl.BlockSpec((D, E), lambda i: (0, 0)),
        ],
        out_specs=[
            pl.BlockSpec((BSQ, D), lambda i: (i, 0)),
            pl.BlockSpec((BSQ, D), lambda i: (i, 0)),
            pl.BlockSpec((BSQ, TOPK), lambda i: (i, 0)),
            pl.BlockSpec((BSQ, TOPK), lambda i: (i, 0)),
            pl.BlockSpec((BSQ, TOPK), lambda i: (i, 0)),
            pl.BlockSpec((1, E), lambda i: (0, 0)),
        ],
        out_shape=[
            jax.ShapeDtypeStruct((S, D), _F32),
            jax.ShapeDtypeStruct((S, D), _F32),
            jax.ShapeDtypeStruct((S, TOPK), _F32),
            jax.ShapeDtypeStruct((S, TOPK), jnp.int32),
            jax.ShapeDtypeStruct((S, TOPK), jnp.int32),
            jax.ShapeDtypeStruct((1, E), jnp.int32),
        ],
        scratch_shapes=[pltpu.VMEM((1, E), _F32)],
    )(h, o, wo, ln2, wg)


def _fixup_body(cnt_ref, eid_ref, rank_ref,
                pos0_out, pos1_out, b_out, e_out, val_out):
    cnt = cnt_ref[...]
    pc = ((cnt + BLK - 1) // BLK) * BLK
    er = jax.lax.broadcasted_iota(jnp.int32, (E, E + 1), 0)
    jc = jax.lax.broadcasted_iota(jnp.int32, (E, E + 1), 1)
    mcum = (er < jc).astype(_F32)
    offp9 = jax.lax.dot(pc.astype(_F32), mcum,
                        precision=_HI).astype(jnp.int32)
    blkb = offp9 // BLK
    nbp = blkb[:, E:]
    g_col = jax.lax.broadcasted_iota(jnp.int32, (G, 1), 0)
    b_of_g = jnp.minimum(g_col, jnp.broadcast_to(nbp, (G, 1)) - 1)
    cmp = (jnp.broadcast_to(blkb, (G, E + 1)) <= b_of_g)
    e_of_g = jnp.clip(jnp.sum(cmp.astype(jnp.int32), axis=-1, keepdims=True)
                      - 1, 0, E - 1)
    b_out[...] = b_of_g
    e_out[...] = e_of_g
    val_out[...] = (g_col < jnp.broadcast_to(nbp, (G, 1))).astype(jnp.int32)

    te = jax.lax.broadcasted_iota(jnp.int32, (S, E), 1)
    off8b = jnp.broadcast_to(offp9[:, :E], (S, E))
    for k, out in ((0, pos0_out), (1, pos1_out)):
        ohk = te == eid_ref[:, k:k + 1]
        offsel = jnp.sum(jnp.where(ohk, off8b, 0), axis=-1, keepdims=True)
        out[...] = offsel + rank_ref[:, k:k + 1]


def _fixup_call(cnt, eid, rank):
    return pl.pallas_call(
        _fixup_body,
        out_shape=[
            jax.ShapeDtypeStruct((S, 1), jnp.int32),
            jax.ShapeDtypeStruct((S, 1), jnp.int32),
            jax.ShapeDtypeStruct((G, 1), jnp.int32),
            jax.ShapeDtypeStruct((G, 1), jnp.int32),
            jax.ShapeDtypeStruct((G, 1), jnp.int32),
        ],
    )(cnt, eid, rank)


def _make_sc_dispatch():
    from jax.experimental.pallas import tpu_sc as plsc
    per_w = S // SC_NW
    CH = 32
    nch = per_w // CH
    mesh = plsc.VectorSubcoreMesh(core_axis_name="c", subcore_axis_name="s",
                                  num_cores=SC_NC)

    @functools.partial(
        pl.kernel, mesh=mesh,
        out_type=jax.ShapeDtypeStruct((A_PAD, D), _F32),
        scratch_types=[
            pltpu.VMEM((CH,), jnp.int32),
            pltpu.VMEM((CH, D), _F32),
            pltpu.SemaphoreType.DMA,
        ],
    )
    def dispatch_k(r2_hbm, p0_hbm, p1_hbm, out_hbm, idx_v, rows_v, sem):
        wid = jax.lax.axis_index("s") * SC_NC + jax.lax.axis_index("c")
        for k in range(TOPK):
            idx_hbm = (p0_hbm, p1_hbm)[k]
            for c in range(nch):
                base = wid * per_w + c * CH
                pltpu.sync_copy(r2_hbm.at[pl.ds(base, CH)], rows_v)
                pltpu.sync_copy(idx_hbm.at[pl.ds(base, CH)], idx_v)
                pltpu.async_copy(rows_v, out_hbm.at[idx_v], sem).wait()

    return dispatch_k


def _make_sc_combine():
    from jax.experimental.pallas import tpu_sc as plsc
    per_w = S // SC_NW
    CH = 32
    nch = per_w // CH
    mesh = plsc.VectorSubcoreMesh(core_axis_name="c", subcore_axis_name="s",
                                  num_cores=SC_NC)

    @functools.partial(
        pl.kernel, mesh=mesh,
        out_type=jax.ShapeDtypeStruct((A, D), _F32),
        scratch_types=[
            pltpu.VMEM((CH,), jnp.int32),
            pltpu.VMEM((CH, D), _F32),
            pltpu.SemaphoreType.DMA,
        ],
    )
    def combine_k(y_hbm, p0_hbm, p1_hbm, out_hbm, idx_v, rows_v, sem):
        wid = jax.lax.axis_index("s") * SC_NC + jax.lax.axis_index("c")
        for k in range(TOPK):
            idx_hbm = (p0_hbm, p1_hbm)[k]
            for c in range(nch):
                base = wid * per_w + c * CH
                pltpu.sync_copy(idx_hbm.at[pl.ds(base, CH)], idx_v)
                pltpu.async_copy(y_hbm.at[idx_v], rows_v, sem).wait()
                pltpu.sync_copy(rows_v, out_hbm.at[pl.ds(k * S + base, CH)])

    return combine_k


_sc_cache = {}


def _sc_dispatch(r2, p0, p1):
    if 'd' not in _sc_cache:
        _sc_cache['d'] = _make_sc_dispatch()
    return _sc_cache['d'](r2, p0, p1)


def _sc_combine(y, p0, p1):
    if 'c' not in _sc_cache:
        _sc_cache['c'] = _make_sc_combine()
    return _sc_cache['c'](y, p0, p1)


def _gmm_body(b_arr, e_arr, val_arr,
              x_ref, w1_ref, w3_ref, w2_ref, out_ref):
    g = pl.program_id(0)

    @pl.when(val_arr[g] > 0)
    def _():
        x = x_ref[...]
        a = jnp.dot(x, w1_ref[0], preferred_element_type=_F32)
        bb = jnp.dot(x, w3_ref[0], preferred_element_type=_F32)
        out_ref[...] = jnp.dot(a * jax.nn.sigmoid(a) * bb, w2_ref[0],
                               preferred_element_type=_F32)


def _gmm_call(xg, w1, w3, w2, b_arr, e_arr, val_arr):
    grid_spec = pltpu.PrefetchScalarGridSpec(
        num_scalar_prefetch=3,
        grid=(G,),
        in_specs=[
            pl.BlockSpec((BLK, D), lambda g, bs, es, vs: (bs[g], 0)),
            pl.BlockSpec((1, D, F), lambda g, bs, es, vs: (es[g], 0, 0)),
            pl.BlockSpec((1, D, F), lambda g, bs, es, vs: (es[g], 0, 0)),
            pl.BlockSpec((1, F, D), lambda g, bs, es, vs: (es[g], 0, 0)),
        ],
        out_specs=pl.BlockSpec((BLK, D), lambda g, bs, es, vs: (bs[g], 0)),
    )
    return pl.pallas_call(
        _gmm_body,
        grid_spec=grid_spec,
        out_shape=jax.ShapeDtypeStruct((A_PAD, D), _F32),
        compiler_params=pltpu.CompilerParams(
            vmem_limit_bytes=100 * 1024 * 1024),
    )(b_arr, e_arr, val_arr, xg, w1, w3, w2)


def _combine_body_plain(h2_ref, ya_ref, yb_ref, tw_ref, out_ref):
    tw = tw_ref[...]
    out_ref[...] = (h2_ref[...] + tw[:, 0:1] * ya_ref[...]
                    + tw[:, 1:2] * yb_ref[...])


def _combine_body_final(h2_ref, ya_ref, yb_ref, tw_ref, fw_ref, out_ref):
    tw = tw_ref[...]
    h3 = (h2_ref[...] + tw[:, 0:1] * ya_ref[...]
          + tw[:, 1:2] * yb_ref[...])
    out_ref[...] = _rms(h3, fw_ref[...])


def _combine_call(h2, yg, tw, final_w=None):
    specs = [
        pl.BlockSpec((BSQ, D), lambda i: (i, 0)),
        pl.BlockSpec((BSQ, D), lambda i: (i, 0)),
        pl.BlockSpec((BSQ, D), lambda i: (S // BSQ + i, 0)),
        pl.BlockSpec((BSQ, TOPK), lambda i: (i, 0)),
    ]
    args = (h2, yg, yg, tw)
    body = _combine_body_plain
    if final_w is not None:
        specs.append(pl.BlockSpec((D,), lambda i: (0,)))
        args = args + (final_w,)
        body = _combine_body_final
    return pl.pallas_call(
        body,
        grid=(NSB,),
        in_specs=specs,
        out_specs=pl.BlockSpec((BSQ, D), lambda i: (i, 0)),
        out_shape=jax.ShapeDtypeStruct((S, D), _F32),
    )(*args)


def _final_body(h_ref, w_ref, out_ref):
    out_ref[...] = _rms(h_ref[...], w_ref[...])


def _final_call(h, w):
    return pl.pallas_call(
        _final_body,
        out_shape=jax.ShapeDtypeStruct((S, D), _F32),
    )(h, w)


def _col_perm_q():
    import numpy as np
    n = np.arange(2 * QW)
    half, rest = n // QW, n % QW
    return (rest // HH) * HD + half * HH + rest % HH


def _col_perm_k():
    import numpy as np
    n = np.arange(2 * KW)
    half, rest = n // KW, n % KW
    return (rest // HH) * HD + half * HH + rest % HH


def _rope_tables():
    inv_freq = 1.0 / (THETA ** (jnp.arange(0, HD, 2).astype(_F32) / HD))
    freqs = jnp.arange(S, dtype=_F32)[:, None] * inv_freq[None, :]
    cosf, sinf = jnp.cos(freqs), jnp.sin(freqs)
    cq = jnp.tile(cosf, (1, 2 * QW // HH))
    sq = jnp.tile(sinf, (1, 2 * QW // HH))
    ck = jnp.tile(cosf, (1, 2 * KW // HH))
    sk = jnp.tile(sinf, (1, 2 * KW // HH))
    return cq, sq, ck, sk


@jax.jit
def _forward(x, params):
    cq, sq, ck, sk = _rope_tables()
    pq, pk = _col_perm_q(), _col_perm_k()
    h = x.reshape(S, D)
    for l in range(L):
        p = params['layer_%d' % l]
        q, k, v = _qkv_call(h, p['ln1'], p['wq'][:, pq], p['wk'][:, pk],
                            p['wv'], cq, sq, ck, sk)
        o = _attn_call(q, k, v)
        h2, r2, tw, eid, rank, cnt = _wo_router_call(
            h, o, p['wo'], p['ln2'], p['wg'])
        pos0, pos1, b_arr, e_arr, val_arr = _fixup_call(cnt, eid, rank)
        p0 = pos0.reshape(S)
        p1 = pos1.reshape(S)
        xg = _sc_dispatch(r2, p0, p1)
        y = _gmm_call(xg, p['w1'], p['w3'], p['w2'],
                      b_arr.reshape(G), e_arr.reshape(G),
                      val_arr.reshape(G))
        yg = _sc_combine(y, p0, p1)
        if l == L - 1:
            h = _combine_call(h2, yg, tw, params['final_ln'])
        else:
            h = _combine_call(h2, yg, tw)
    return h.reshape(1, S, D)


def kernel(input_ids, params):
    return _forward(input_ids, params)


# speedup vs baseline: 1.7434x; 1.0134x over previous
import functools

import jax
import jax.numpy as jnp
from jax.experimental import pallas as pl
from jax.experimental.pallas import tpu as pltpu

S, D = 2048, 1024
H, KV, HD = 16, 8, 64
E, TOPK, F = 8, 2, 1024
L = 2
EPS = 1e-6
THETA = 10000.0
HH = HD // 2

QW = H * HH
KW = KV * HH

BSQ = 512
NSB = S // BSQ

A = TOPK * S
BLK = 640
G = (E - 1) + (A - (E - 1) + BLK - 1) // BLK
NB = G
A_PAD = G * BLK

SC_NC, SC_NS = 2, 16
SC_NW = SC_NC * SC_NS

_F32 = jnp.float32
_HI = jax.lax.Precision.HIGHEST


def _rms(x, w):
    return x * jax.lax.rsqrt(jnp.mean(x * x, axis=-1, keepdims=True) + EPS) * w


def _qkv_body(h_ref, ln1_ref, wq_ref, wk_ref, wv_ref, cq_ref, sq_ref,
              ck_ref, sk_ref, q_out, k_out, v_out):
    r = _rms(h_ref[...], ln1_ref[...])
    q = jnp.dot(r, wq_ref[...], preferred_element_type=_F32)
    k = jnp.dot(r, wk_ref[...], preferred_element_type=_F32)
    v = jnp.dot(r, wv_ref[...], preferred_element_type=_F32)
    qr = jnp.concatenate([-q[:, QW:], q[:, :QW]], axis=1)
    kr = jnp.concatenate([-k[:, KW:], k[:, :KW]], axis=1)
    q_out[...] = q * cq_ref[...] + qr * sq_ref[...]
    k_out[...] = k * ck_ref[...] + kr * sk_ref[...]
    v_out[...] = v


def _qkv_call(h, ln1, wq_p, wk_p, wv, cq, sq, ck, sk):
    return pl.pallas_call(
        _qkv_body,
        grid=(NSB,),
        in_specs=[
            pl.BlockSpec((BSQ, D), lambda i: (i, 0)),
            pl.BlockSpec((D,), lambda i: (0,)),
            pl.BlockSpec((D, 2 * QW), lambda i: (0, 0)),
            pl.BlockSpec((D, 2 * KW), lambda i: (0, 0)),
            pl.BlockSpec((D, KV * HD), lambda i: (0, 0)),
            pl.BlockSpec((BSQ, 2 * QW), lambda i: (i, 0)),
            pl.BlockSpec((BSQ, 2 * QW), lambda i: (i, 0)),
            pl.BlockSpec((BSQ, 2 * KW), lambda i: (i, 0)),
            pl.BlockSpec((BSQ, 2 * KW), lambda i: (i, 0)),
        ],
        out_specs=[
            pl.BlockSpec((BSQ, 2 * QW), lambda i: (i, 0)),
            pl.BlockSpec((BSQ, 2 * KW), lambda i: (i, 0)),
            pl.BlockSpec((BSQ, KV * HD), lambda i: (i, 0)),
        ],
        out_shape=[
            jax.ShapeDtypeStruct((S, 2 * QW), _F32),
            jax.ShapeDtypeStruct((S, 2 * KW), _F32),
            jax.ShapeDtypeStruct((S, KV * HD), _F32),
        ],
    )(h, ln1, wq_p, wk_p, wv, cq, sq, ck, sk)


BQ = 256
NQB = S // BQ


def _make_attn_body(qb0, kwid):
    def body(q_ref, k_ref, v_ref, o_ref):
        qb_i = pl.program_id(0)
        q = q_ref[...]
        k = k_ref[...]
        v = v_ref[...]
        rows = (jax.lax.broadcasted_iota(jnp.int32, (BQ, kwid), 0)
                + (qb0 + qb_i) * BQ)
        cols = jax.lax.broadcasted_iota(jnp.int32, (BQ, kwid), 1)
        bias = jnp.where(cols <= rows, 0.0, -1e9)
        for h in range(H):
            j = h // 2
            qh = jnp.concatenate(
                [q[:, h * HH:(h + 1) * HH],
                 q[:, QW + h * HH:QW + (h + 1) * HH]], axis=1)
            kh = jnp.concatenate(
                [k[:, j * HH:(j + 1) * HH],
                 k[:, KW + j * HH:KW + (j + 1) * HH]], axis=1)
            s = jnp.dot(qh, kh.T, preferred_element_type=_F32)
            s = s * (1.0 / (HD ** 0.5)) + bias
            p = jax.nn.softmax(s, axis=-1)
            o_ref[:, h * HD:(h + 1) * HD] = jnp.dot(
                p, v[:, j * HD:(j + 1) * HD], preferred_element_type=_F32)
    return body


QG = 2


def _attn_call(q, k, v):
    outs = []
    for g in range(NQB // QG):
        qb0 = g * QG
        kwid = (qb0 + QG) * BQ
        o_g = pl.pallas_call(
            _make_attn_body(qb0, kwid),
            grid=(QG,),
            in_specs=[
                pl.BlockSpec((BQ, 2 * QW), lambda qb, qb0=qb0: (qb0 + qb, 0)),
                pl.BlockSpec((kwid, 2 * KW), lambda qb: (0, 0)),
                pl.BlockSpec((kwid, KV * HD), lambda qb: (0, 0)),
            ],
            out_specs=pl.BlockSpec((BQ, H * HD), lambda qb: (qb, 0)),
            out_shape=jax.ShapeDtypeStruct((QG * BQ, H * HD), _F32),
            compiler_params=pltpu.CompilerParams(
                vmem_limit_bytes=100 * 1024 * 1024),
        )(q, k, v)
        outs.append(o_g)
    return jnp.concatenate(outs, axis=0)


def _wo_router_body(h_ref, o_ref, wo_ref, ln2_ref, wg_ref,
                    h2_out, r2_out, tw_out, eid_out, rank_out, cnt_out,
                    run_ref):
    i = pl.program_id(0)

    @pl.when(i == 0)
    def _():
        run_ref[...] = jnp.zeros((1, E), _F32)

    h2 = h_ref[...] + jnp.dot(o_ref[...], wo_ref[...],
                              preferred_element_type=_F32)
    h2_out[...] = h2
    r2 = _rms(h2, ln2_ref[...])
    r2_out[...] = r2
    logits = jnp.dot(r2, wg_ref[...], preferred_element_type=_F32)
    probs = jax.nn.softmax(logits, axis=-1)
    idx = jax.lax.broadcasted_iota(jnp.int32, (BSQ, E), 1)
    m1 = jnp.max(probs, axis=-1, keepdims=True)
    i1 = jnp.min(jnp.where(probs == m1, idx, E), axis=-1, keepdims=True)
    oh1 = (idx == i1).astype(_F32)
    rest = jnp.where(idx == i1, -jnp.inf, probs)
    m2 = jnp.max(rest, axis=-1, keepdims=True)
    i2 = jnp.min(jnp.where(rest == m2, idx, E), axis=-1, keepdims=True)
    oh2 = (idx == i2).astype(_F32)
    denom = m1 + m2
    tw_out[...] = jnp.concatenate([m1 / denom, m2 / denom], axis=1)
    eid_out[...] = jnp.concatenate([i1, i2], axis=1)

    oh12 = oh1 + oh2
    tr = jax.lax.broadcasted_iota(jnp.int32, (BSQ, BSQ), 0)
    tc = jax.lax.broadcasted_iota(jnp.int32, (BSQ, BSQ), 1)
    strict = (tc < tr).astype(_F32)
    pref = jax.lax.dot(strict, oh12, precision=_HI) + run_ref[...]
    rank1 = jnp.sum(oh1 * pref, axis=-1, keepdims=True)
    rank2 = jnp.sum(oh2 * pref, axis=-1, keepdims=True)
    rank_out[...] = jnp.concatenate([rank1, rank2], axis=1).astype(jnp.int32)
    run_ref[...] += jnp.sum(oh12, axis=0, keepdims=True)

    @pl.when(i == NSB - 1)
    def _():
        cnt_out[...] = run_ref[...].astype(jnp.int32)


def _wo_router_call(h, o, wo, ln2, wg):
    return pl.pallas_call(
        _wo_router_body,
        grid=(NSB,),
        in_specs=[
            pl.BlockSpec((BSQ, D), lambda i: (i, 0)),
            pl.BlockSpec((BSQ, H * HD), lambda i: (i, 0)),
            pl.BlockSpec((H * HD, D), lambda i: (0, 0)),
            pl.BlockSpec((D,), lambda i: (0,)),
            pl.BlockSpec((D, E), lambda i: (0, 0)),
        ],
        out_specs=[
            pl.BlockSpec((BSQ, D), lambda i: (i, 0)),
            pl.BlockSpec((BSQ, D), lambda i: (i, 0)),
            pl.BlockSpec((BSQ, TOPK), lambda i: (i, 0)),
            pl.BlockSpec((BSQ, TOPK), lambda i: (i, 0)),
            pl.BlockSpec((BSQ, TOPK), lambda i: (i, 0)),
            pl.BlockSpec((1, E), lambda i: (0, 0)),
        ],
        out_shape=[
            jax.ShapeDtypeStruct((S, D), _F32),
            jax.ShapeDtypeStruct((S, D), _F32),
            jax.ShapeDtypeStruct((S, TOPK), _F32),
            jax.ShapeDtypeStruct((S, TOPK), jnp.int32),
            jax.ShapeDtypeStruct((S, TOPK), jnp.int32),
            jax.ShapeDtypeStruct((1, E), jnp.int32),
        ],
        scratch_shapes=[pltpu.VMEM((1, E), _F32)],
    )(h, o, wo, ln2, wg)


def _fixup_body(cnt_ref, eid_ref, rank_ref,
                pos0_out, pos1_out, b_out, e_out, val_out):
    cnt = cnt_ref[...]
    pc = ((cnt + BLK - 1) // BLK) * BLK
    er = jax.lax.broadcasted_iota(jnp.int32, (E, E + 1), 0)
    jc = jax.lax.broadcasted_iota(jnp.int32, (E, E + 1), 1)
    mcum = (er < jc).astype(_F32)
    offp9 = jax.lax.dot(pc.astype(_F32), mcum,
                        precision=_HI).astype(jnp.int32)
    blkb = offp9 // BLK
    nbp = blkb[:, E:]
    g_col = jax.lax.broadcasted_iota(jnp.int32, (G, 1), 0)
    b_of_g = jnp.minimum(g_col, jnp.broadcast_to(nbp, (G, 1)) - 1)
    cmp = (jnp.broadcast_to(blkb, (G, E + 1)) <= b_of_g)
    e_of_g = jnp.clip(jnp.sum(cmp.astype(jnp.int32), axis=-1, keepdims=True)
                      - 1, 0, E - 1)
    b_out[...] = b_of_g
    e_out[...] = e_of_g
    val_out[...] = (g_col < jnp.broadcast_to(nbp, (G, 1))).astype(jnp.int32)

    te = jax.lax.broadcasted_iota(jnp.int32, (S, E), 1)
    off8b = jnp.broadcast_to(offp9[:, :E], (S, E))
    for k, out in ((0, pos0_out), (1, pos1_out)):
        ohk = te == eid_ref[:, k:k + 1]
        offsel = jnp.sum(jnp.where(ohk, off8b, 0), axis=-1, keepdims=True)
        out[...] = offsel + rank_ref[:, k:k + 1]


def _fixup_call(cnt, eid, rank):
    return pl.pallas_call(
        _fixup_body,
        out_shape=[
            jax.ShapeDtypeStruct((S, 1), jnp.int32),
            jax.ShapeDtypeStruct((S, 1), jnp.int32),
            jax.ShapeDtypeStruct((G, 1), jnp.int32),
            jax.ShapeDtypeStruct((G, 1), jnp.int32),
            jax.ShapeDtypeStruct((G, 1), jnp.int32),
        ],
    )(cnt, eid, rank)


def _make_sc_dispatch():
    from jax.experimental.pallas import tpu_sc as plsc
    per_w = S // SC_NW
    CH = 32
    nch = per_w // CH
    mesh = plsc.VectorSubcoreMesh(core_axis_name="c", subcore_axis_name="s",
                                  num_cores=SC_NC)

    @functools.partial(
        pl.kernel, mesh=mesh,
        out_type=jax.ShapeDtypeStruct((A_PAD, D), _F32),
        scratch_types=[
            pltpu.VMEM((CH,), jnp.int32),
            pltpu.VMEM((CH, D), _F32),
            pltpu.SemaphoreType.DMA,
        ],
    )
    def dispatch_k(r2_hbm, p0_hbm, p1_hbm, out_hbm, idx_v, rows_v, sem):
        wid = jax.lax.axis_index("s") * SC_NC + jax.lax.axis_index("c")
        for k in range(TOPK):
            idx_hbm = (p0_hbm, p1_hbm)[k]
            for c in range(nch):
                base = wid * per_w + c * CH
                pltpu.sync_copy(r2_hbm.at[pl.ds(base, CH)], rows_v)
                pltpu.sync_copy(idx_hbm.at[pl.ds(base, CH)], idx_v)
                pltpu.async_copy(rows_v, out_hbm.at[idx_v], sem).wait()

    return dispatch_k


def _make_sc_combine():
    from jax.experimental.pallas import tpu_sc as plsc
    per_w = S // SC_NW
    CH = 32
    nch = per_w // CH
    mesh = plsc.VectorSubcoreMesh(core_axis_name="c", subcore_axis_name="s",
                                  num_cores=SC_NC)

    @functools.partial(
        pl.kernel, mesh=mesh,
        out_type=jax.ShapeDtypeStruct((A, D), _F32),
        scratch_types=[
            pltpu.VMEM((CH,), jnp.int32),
            pltpu.VMEM((CH, D), _F32),
            pltpu.SemaphoreType.DMA,
        ],
    )
    def combine_k(y_hbm, p0_hbm, p1_hbm, out_hbm, idx_v, rows_v, sem):
        wid = jax.lax.axis_index("s") * SC_NC + jax.lax.axis_index("c")
        for k in range(TOPK):
            idx_hbm = (p0_hbm, p1_hbm)[k]
            for c in range(nch):
                base = wid * per_w + c * CH
                pltpu.sync_copy(idx_hbm.at[pl.ds(base, CH)], idx_v)
                pltpu.async_copy(y_hbm.at[idx_v], rows_v, sem).wait()
                pltpu.sync_copy(rows_v, out_hbm.at[pl.ds(k * S + base, CH)])

    return combine_k


_sc_cache = {}


def _sc_dispatch(r2, p0, p1):
    if 'd' not in _sc_cache:
        _sc_cache['d'] = _make_sc_dispatch()
    return _sc_cache['d'](r2, p0, p1)


def _sc_combine(y, p0, p1):
    if 'c' not in _sc_cache:
        _sc_cache['c'] = _make_sc_combine()
    return _sc_cache['c'](y, p0, p1)


def _gmm_body(b_arr, e_arr, val_arr,
              x_ref, w1_ref, w3_ref, w2_ref, out_ref):
    g = pl.program_id(0)

    @pl.when(val_arr[g] > 0)
    def _():
        x = x_ref[...]
        a = jnp.dot(x, w1_ref[0], preferred_element_type=_F32)
        bb = jnp.dot(x, w3_ref[0], preferred_element_type=_F32)
        out_ref[...] = jnp.dot(a * jax.nn.sigmoid(a) * bb, w2_ref[0],
                               preferred_element_type=_F32)


def _gmm_call(xg, w1, w3, w2, b_arr, e_arr, val_arr):
    grid_spec = pltpu.PrefetchScalarGridSpec(
        num_scalar_prefetch=3,
        grid=(G,),
        in_specs=[
            pl.BlockSpec((BLK, D), lambda g, bs, es, vs: (bs[g], 0)),
            pl.BlockSpec((1, D, F), lambda g, bs, es, vs: (es[g], 0, 0)),
            pl.BlockSpec((1, D, F), lambda g, bs, es, vs: (es[g], 0, 0)),
            pl.BlockSpec((1, F, D), lambda g, bs, es, vs: (es[g], 0, 0)),
        ],
        out_specs=pl.BlockSpec((BLK, D), lambda g, bs, es, vs: (bs[g], 0)),
    )
    return pl.pallas_call(
        _gmm_body,
        grid_spec=grid_spec,
        out_shape=jax.ShapeDtypeStruct((A_PAD, D), _F32),
        compiler_params=pltpu.CompilerParams(
            vmem_limit_bytes=100 * 1024 * 1024),
    )(b_arr, e_arr, val_arr, xg, w1, w3, w2)


def _combine_body_plain(h2_ref, ya_ref, yb_ref, tw_ref, out_ref):
    tw = tw_ref[...]
    out_ref[...] = (h2_ref[...] + tw[:, 0:1] * ya_ref[...]
                    + tw[:, 1:2] * yb_ref[...])


def _combine_body_final(h2_ref, ya_ref, yb_ref, tw_ref, fw_ref, out_ref):
    tw = tw_ref[...]
    h3 = (h2_ref[...] + tw[:, 0:1] * ya_ref[...]
          + tw[:, 1:2] * yb_ref[...])
    out_ref[...] = _rms(h3, fw_ref[...])


def _combine_call(h2, yg, tw, final_w=None):
    specs = [
        pl.BlockSpec((BSQ, D), lambda i: (i, 0)),
        pl.BlockSpec((BSQ, D), lambda i: (i, 0)),
        pl.BlockSpec((BSQ, D), lambda i: (S // BSQ + i, 0)),
        pl.BlockSpec((BSQ, TOPK), lambda i: (i, 0)),
    ]
    args = (h2, yg, yg, tw)
    body = _combine_body_plain
    if final_w is not None:
        specs.append(pl.BlockSpec((D,), lambda i: (0,)))
        args = args + (final_w,)
        body = _combine_body_final
    return pl.pallas_call(
        body,
        grid=(NSB,),
        in_specs=specs,
        out_specs=pl.BlockSpec((BSQ, D), lambda i: (i, 0)),
        out_shape=jax.ShapeDtypeStruct((S, D), _F32),
    )(*args)


def _final_body(h_ref, w_ref, out_ref):
    out_ref[...] = _rms(h_ref[...], w_ref[...])


def _final_call(h, w):
    return pl.pallas_call(
        _final_body,
        out_shape=jax.ShapeDtypeStruct((S, D), _F32),
    )(h, w)


def _col_perm_q():
    import numpy as np
    n = np.arange(2 * QW)
    half, rest = n // QW, n % QW
    return (rest // HH) * HD + half * HH + rest % HH


def _col_perm_k():
    import numpy as np
    n = np.arange(2 * KW)
    half, rest = n // KW, n % KW
    return (rest // HH) * HD + half * HH + rest % HH


def _rope_tables():
    inv_freq = 1.0 / (THETA ** (jnp.arange(0, HD, 2).astype(_F32) / HD))
    freqs = jnp.arange(S, dtype=_F32)[:, None] * inv_freq[None, :]
    cosf, sinf = jnp.cos(freqs), jnp.sin(freqs)
    cq = jnp.tile(cosf, (1, 2 * QW // HH))
    sq = jnp.tile(sinf, (1, 2 * QW // HH))
    ck = jnp.tile(cosf, (1, 2 * KW // HH))
    sk = jnp.tile(sinf, (1, 2 * KW // HH))
    return cq, sq, ck, sk


@jax.jit
def _forward(x, params):
    cq, sq, ck, sk = _rope_tables()
    pq, pk = _col_perm_q(), _col_perm_k()
    h = x.reshape(S, D)
    for l in range(L):
        p = params['layer_%d' % l]
        q, k, v = _qkv_call(h, p['ln1'], p['wq'][:, pq], p['wk'][:, pk],
                            p['wv'], cq, sq, ck, sk)
        o = _attn_call(q, k, v)
        h2, r2, tw, eid, rank, cnt = _wo_router_call(
            h, o, p['wo'], p['ln2'], p['wg'])
        pos0, pos1, b_arr, e_arr, val_arr = _fixup_call(cnt, eid, rank)
        p0 = pos0.reshape(S)
        p1 = pos1.reshape(S)
        xg = _sc_dispatch(r2, p0, p1)
        y = _gmm_call(xg, p['w1'], p['w3'], p['w2'],
                      b_arr.reshape(G), e_arr.reshape(G),
                      val_arr.reshape(G))
        yg = _sc_combine(y, p0, p1)
        if l == L - 1:
            h = _combine_call(h2, yg, tw, params['final_ln'])
        else:
            h = _combine_call(h2, yg, tw)
    return h.reshape(1, S, D)


def kernel(input_ids, params):
    return _forward(input_ids, params)
